# baseline (speedup 1.0000x reference)
import functools

import jax
import jax.numpy as jnp
from jax import lax
from jax.experimental import pallas as pl
from jax.experimental.pallas import tpu as pltpu
from jax.experimental.pallas import tpu_sc as plsc

N = 10000
E = 320000
H = 128
C = 10
G = 128

NP = 10240
NPZ = 10112
K = 80
NCH = 126
NT = 32
NCHW = 2 * NCH
SUP = 6
SUPW = 8
NSUPW = 32
ND = 4
RB = 1024
NBLK = NP // RB
RPT = NP // 16
RPZ = NPZ // 16

f32 = jnp.float32
i32 = jnp.int32


def _rowmask(i):
    rows = lax.broadcasted_iota(i32, (RB, 1), 0) + i * RB
    return (rows < N).astype(f32)


def _affine(s_ref):
    s0 = s_ref[0:1, :]
    s1 = s_ref[1:2, :]
    m = s0 * (1.0 / N)
    v = s1 * (1.0 / N) - m * m
    al = lax.rsqrt(v + 1e-5)
    return al, -m * al + 1e-4


def _stats_of(x, i, s_ref):
    xm = x * _rowmask(i)
    part = jnp.concatenate(
        [jnp.sum(xm, 0, keepdims=True), jnp.sum(xm * xm, 0, keepdims=True),
         jnp.zeros((6, H), f32)], axis=0)

    @pl.when(i == 0)
    def _():
        s_ref[...] = part

    @pl.when(i > 0)
    def _():
        s_ref[...] = s_ref[...] + part


def _tc_stats_body(x_ref, s_ref):
    _stats_of(x_ref[...], pl.program_id(0), s_ref)


def _tc_feat_body(x_ref, sx_ref, Wf_ref, bf_ref, h_ref, s1_ref):
    i = pl.program_id(0)
    al, bt = _affine(sx_ref)
    xn = x_ref[...] * al + bt
    h = jnp.maximum(jnp.dot(xn, Wf_ref[...], preferred_element_type=f32)
                    + bf_ref[...], 0.0)
    h_ref[...] = h
    _stats_of(h, i, s1_ref)


def _tc_front_body(h_ref, sh_ref, deg_ref, W_ref, xw_ref, y_ref):
    i = pl.program_id(0)
    al, bt = _affine(sh_ref)
    xn = h_ref[...] * al + bt
    xw = jnp.dot(xn, W_ref[...], preferred_element_type=f32)
    dis = lax.rsqrt(deg_ref[0] + deg_ref[1] + 1.0)
    xw_ref[...] = xw
    y_ref[...] = dis * xw * _rowmask(i)


def _tc_back_body(z_ref, xw_ref, deg_ref, b_ref, h_ref, s_ref):
    i = pl.program_id(0)
    dis = lax.rsqrt(deg_ref[0] + deg_ref[1] + 1.0)
    xw = xw_ref[...]
    h = jnp.maximum(dis * (z_ref[0] + z_ref[1]) + dis * dis * xw + b_ref[...],
                    0.0)
    h_ref[...] = h
    _stats_of(h, i, s_ref)


def _tc_att_body(h_ref, We_ref, be_ref, Wn_ref, bnb_ref,
                 u_ref, v_ref, xc_ref, xo_ref, sc_ref, so_ref):
    i = pl.program_id(0)
    hb = h_ref[...]
    lanes = lax.broadcasted_iota(i32, (1, H), 1)
    sel = jnp.where(lanes == 0, 1.0, jnp.where(lanes == 1, -1.0, 0.0))
    wuv = lax.dot_general(sel, We_ref[...], (((1,), (1,)), ((), ())),
                          preferred_element_type=f32)
    wu = wuv[:, 0:H]
    wv = wuv[:, H:2 * H]
    be_d = jnp.sum(be_ref[...] * sel, axis=1, keepdims=True)
    urows = []
    vrows = []
    for sb in range(RB // 128):
        hs = hb[sb * 128:(sb + 1) * 128, :]
        urows.append(lax.dot_general(wu, hs, (((1,), (1,)), ((), ())),
                                     preferred_element_type=f32) + be_d)
        vrows.append(lax.dot_general(wv, hs, (((1,), (1,)), ((), ())),
                                     preferred_element_type=f32))
    u_ref[...] = jnp.concatenate(urows, axis=0)
    v_ref[...] = jnp.concatenate(vrows, axis=0)
    nl = jnp.dot(hb, Wn_ref[...], preferred_element_type=f32) + bnb_ref[...]
    d01 = nl[:, 0:1] - nl[:, 1:2]
    att0 = 1.0 / (1.0 + jnp.exp(-d01))
    xc = att0 * hb
    xo = hb - xc
    xc_ref[...] = xc
    xo_ref[...] = xo
    _stats_of(xc, i, sc_ref)
    _stats_of(xo, i, so_ref)


def _split_dis(dgw_ref):
    dsum = dgw_ref[0] + dgw_ref[1]
    ri = lax.broadcasted_iota(i32, (H, H), 0)
    s0 = (ri == 0).astype(f32)
    s64 = (ri == 64).astype(f32)
    degc = jnp.dot(dsum, s0, preferred_element_type=f32)
    dego = jnp.dot(dsum, s64, preferred_element_type=f32)
    return lax.rsqrt(degc + 1.0), lax.rsqrt(dego + 1.0)


def _tc_wfront_body(xc_ref, xo_ref, sc_ref, so_ref, dgw_ref,
                    Wc_ref, Wo_ref, xw_ref, y_ref):
    i = pl.program_id(0)
    msk = _rowmask(i)
    disc, diso = _split_dis(dgw_ref)
    alc, btc = _affine(sc_ref)
    xwc = jnp.dot(xc_ref[...] * alc + btc, Wc_ref[...],
                  preferred_element_type=f32)
    alo, bto = _affine(so_ref)
    xwo = jnp.dot(xo_ref[...] * alo + bto, Wo_ref[...],
                  preferred_element_type=f32)
    xw_ref[0] = xwc
    xw_ref[1] = xwo
    y_ref[0] = disc * xwc * msk
    y_ref[1] = diso * xwo * msk


def _tc_final_body(zw_ref, xw_ref, dgw_ref, bc_ref, bo_ref, oh_ref,
                   pc_ref, po_ref):
    i = pl.program_id(0)
    disc, diso = _split_dis(dgw_ref)
    xc2 = jnp.maximum(disc * zw_ref[0] + disc * disc * xw_ref[0] + bc_ref[...],
                      0.0)
    xo2 = jnp.maximum(diso * zw_ref[1] + diso * diso * xw_ref[1] + bo_ref[...],
                      0.0)
    oh = oh_ref[...]
    pc = lax.dot_general(oh, xc2, (((0,), (0,)), ((), ())),
                         preferred_element_type=f32)
    po = lax.dot_general(oh, xo2, (((0,), (0,)), ((), ())),
                         preferred_element_type=f32)

    @pl.when(i == 0)
    def _():
        pc_ref[...] = pc
        po_ref[...] = po

    @pl.when(i > 0)
    def _():
        pc_ref[...] = pc_ref[...] + pc
        po_ref[...] = po_ref[...] + po


def _gstats(x):
    s0 = jnp.sum(x, 0, keepdims=True)
    s1 = jnp.sum(x * x, 0, keepdims=True)
    m = s0 * (1.0 / G)
    v = s1 * (1.0 / G) - m * m
    al = lax.rsqrt(v + 1e-5)
    return al, -m * al + 1e-4


def _logsm(lo):
    lanes = lax.broadcasted_iota(i32, (G, H), 1)
    lom = jnp.where(lanes < C, lo, -1e30)
    mx = jnp.max(lom, axis=1, keepdims=True)
    ls = jnp.log(jnp.sum(jnp.exp(lom - mx), axis=1, keepdims=True))
    return lo - mx - ls


def _tc_readout_body(xcg_ref, xog_ref, W1c_ref, b1c_ref, W2c_ref, b2c_ref,
                     W1o_ref, b1o_ref, W2o_ref, b2o_ref,
                     W1t_ref, W1b_ref, b1co_ref, W2co_ref, b2co_ref,
                     oc_ref, oo_ref, oco_ref):
    xcg = xcg_ref[...]
    xog = xog_ref[...]

    def head(xg, W1r, b1r, W2r, b2r):
        al, bt = _gstats(xg)
        hh = jnp.maximum(jnp.dot(xg * al + bt, W1r,
                                 preferred_element_type=f32) + b1r, 0.0)
        al2, bt2 = _gstats(hh)
        return _logsm(jnp.dot(hh * al2 + bt2, W2r,
                              preferred_element_type=f32) + b2r)

    oc_ref[...] = head(xcg, W1c_ref[...], b1c_ref[...], W2c_ref[...],
                       b2c_ref[...])
    oo_ref[...] = head(xog, W1o_ref[...], b1o_ref[...], W2o_ref[...],
                       b2o_ref[...])
    alc, btc = _gstats(xcg)
    alo, bto = _gstats(xog)
    hh = jnp.maximum(
        jnp.dot(xcg * alc + btc, W1t_ref[...], preferred_element_type=f32)
        + jnp.dot(xog * alo + bto, W1b_ref[...], preferred_element_type=f32)
        + b1co_ref[...], 0.0)
    al2, bt2 = _gstats(hh)
    oco_ref[...] = _logsm(jnp.dot(hh * al2 + bt2, W2co_ref[...],
                                  preferred_element_type=f32) + b2co_ref[...])


_B_NH = pl.BlockSpec((RB, H), lambda i: (i, 0))
_B_2NH = pl.BlockSpec((2, RB, H), lambda i: (0, i, 0))
_B_S = pl.BlockSpec((8, H), lambda i: (0, 0))
_B_W = pl.BlockSpec((H, H), lambda i: (0, 0))
_B_B = pl.BlockSpec((1, H), lambda i: (0, 0))
_B_U = pl.BlockSpec((RB // 128, H), lambda i: (i, 0))
_B_G = pl.BlockSpec((G, H), lambda i: (0, 0))
_SNH = jax.ShapeDtypeStruct((NP, H), f32)
_S2NH = jax.ShapeDtypeStruct((2, NP, H), f32)
_SS = jax.ShapeDtypeStruct((8, H), f32)
_SU = jax.ShapeDtypeStruct((NP // H, H), f32)
_SG = jax.ShapeDtypeStruct((G, H), f32)


def _tc_stats(x):
    return pl.pallas_call(_tc_stats_body, grid=(NBLK,), in_specs=[_B_NH],
                          out_specs=_B_S, out_shape=_SS)(x)


def _tc_feat(x, sx, Wf, bf):
    return pl.pallas_call(
        _tc_feat_body, grid=(NBLK,),
        in_specs=[_B_NH, _B_S, _B_W, _B_B],
        out_specs=[_B_NH, _B_S], out_shape=[_SNH, _SS])(x, sx, Wf, bf)


def _tc_front(h, sh, degb, W):
    return pl.pallas_call(
        _tc_front_body, grid=(NBLK,),
        in_specs=[_B_NH, _B_S, _B_2NH, _B_W],
        out_specs=[_B_NH, _B_NH], out_shape=[_SNH, _SNH])(h, sh, degb, W)


def _tc_back(z, xw, degb, b):
    return pl.pallas_call(
        _tc_back_body, grid=(NBLK,),
        in_specs=[_B_2NH, _B_NH, _B_2NH, _B_B],
        out_specs=[_B_NH, _B_S], out_shape=[_SNH, _SS])(z, xw, degb, b)


def _tc_att(h, We_p, be_p, Wn_p, bnb_p):
    return pl.pallas_call(
        _tc_att_body, grid=(NBLK,),
        in_specs=[_B_NH, pl.BlockSpec((2 * H, H), lambda i: (0, 0)), _B_B,
                  _B_W, _B_B],
        out_specs=[_B_U, _B_U, _B_NH, _B_NH, _B_S, _B_S],
        out_shape=[_SU, _SU, _SNH, _SNH, _SS, _SS])(h, We_p, be_p, Wn_p, bnb_p)


def _tc_wfront(xc, xo, sc, so, dgw, Wc, Wo):
    return pl.pallas_call(
        _tc_wfront_body, grid=(NBLK,),
        in_specs=[_B_NH, _B_NH, _B_S, _B_S, _B_2NH, _B_W, _B_W],
        out_specs=[_B_2NH, _B_2NH],
        out_shape=[_S2NH, _S2NH])(xc, xo, sc, so, dgw, Wc, Wo)


def _tc_final(zw, xw, dgw, bc, bo, oh):
    return pl.pallas_call(
        _tc_final_body, grid=(NBLK,),
        in_specs=[_B_2NH, _B_2NH, _B_2NH, _B_B, _B_B,
                  pl.BlockSpec((RB, G), lambda i: (i, 0))],
        out_specs=[_B_G, _B_G], out_shape=[_SG, _SG])(zw, xw, dgw, bc, bo, oh)


def _tc_readout(xcg, xog, ws):
    gspec = pl.BlockSpec((G, H), lambda: (0, 0))
    bspec = pl.BlockSpec((1, H), lambda: (0, 0))
    specs = [gspec, gspec] + [gspec if w.shape[0] == H else bspec for w in ws]
    return pl.pallas_call(
        _tc_readout_body, grid=(),
        in_specs=specs, out_specs=[gspec, gspec, gspec],
        out_shape=[_SG, _SG, _SG])(xcg, xog, *ws)


def _sc_hist_body(row_hbm, ones_hbm, zeros_hbm, deg_out,
                  row_v, ones_v, deg_sh):
    cid = lax.axis_index("c")
    sid = lax.axis_index("s")
    wid = cid * 16 + sid
    r0 = sid * RPT
    pltpu.sync_copy(row_hbm.at[wid], row_v)
    pltpu.sync_copy(ones_hbm, ones_v)
    pltpu.sync_copy(zeros_hbm.at[pl.ds(r0, RPT)], deg_sh.at[pl.ds(r0, RPT)])
    plsc.subcore_barrier()

    def step(jb, c):
        pltpu.sync_copy(ones_v, deg_sh.at[row_v.at[jb]], add=True)
        return c

    lax.fori_loop(0, NCH, step, 0)
    plsc.subcore_barrier()
    pltpu.sync_copy(deg_sh.at[pl.ds(r0, RPT)], deg_out.at[cid, pl.ds(r0, RPT)])


def _sc_spmm_body(y_hbm, rc_hbm, zeros_hbm, z_out, IS, D, z_sh,
                  i0, i1, g0, g1, g2, g3, s0, s1, s2, s3):
    cid = lax.axis_index("c")
    sid = lax.axis_index("s")
    wid = cid * 16 + sid
    r0 = sid * RPZ
    isems = (i0, i1)
    gsems = (g0, g1, g2, g3)
    ssems = (s0, s1, s2, s3)
    NSUP = NCH // SUP

    def idx_dma(s):
        pltpu.async_copy(rc_hbm.at[wid, pl.ds(s * SUP, SUP)], IS.at[s % 2],
                         isems[s % 2])

    def wait_i(s):
        pltpu.make_async_copy(rc_hbm.at[0, pl.ds(0, SUP)], IS.at[s % 2],
                              isems[s % 2]).wait()

    def gather(c, slot):
        s, j = divmod(c, SUP)
        pltpu.async_copy(y_hbm.at[IS.at[s % 2, j, 0]], D.at[slot],
                         gsems[slot])

    def wait_g(slot):
        pltpu.make_async_copy(y_hbm.at[IS.at[0, 0, 0]], D.at[slot],
                              gsems[slot]).wait()

    def scatter(c):
        s, j = divmod(c, SUP)
        pltpu.async_copy(D.at[c % ND], z_sh.at[IS.at[s % 2, j, 1]],
                         ssems[c % ND], add=True)

    def wait_s(slot):
        pltpu.make_async_copy(D.at[slot], z_sh.at[IS.at[0, 0, 1]],
                              ssems[slot]).wait()

    idx_dma(0)
    pltpu.sync_copy(zeros_hbm.at[pl.ds(r0, RPZ)], z_sh.at[pl.ds(r0, RPZ)])
    plsc.subcore_barrier()
    wait_i(0)
    gather(0, 0)
    gather(1, 1)
    swaited = set()

    def scatter_done(cc):
        if cc >= 0 and cc not in swaited:
            wait_s(cc % ND)
            swaited.add(cc)

    for c in range(NCH):
        s, j = divmod(c, SUP)
        if j == 0 and s + 1 < NSUP:
            scatter_done(c - 2)
            scatter_done(c - 1)
            idx_dma(s + 1)
        if j == SUP - 2 and s + 1 < NSUP:
            wait_i(s + 1)
        scatter_done(c - 2)
        gather(min(c + 2, NCH - 1), (c + 2) % ND)
        wait_g(c % ND)
        scatter(c)
    wait_g((NCH) % ND)
    wait_g((NCH + 1) % ND)
    scatter_done(NCH - 2)
    scatter_done(NCH - 1)
    plsc.subcore_barrier()
    pltpu.sync_copy(z_sh.at[pl.ds(r0, RPZ)], z_out.at[cid, pl.ds(r0, RPZ)])

    @pl.when(sid == 15)
    def _():
        pltpu.sync_copy(zeros_hbm.at[pl.ds(0, NP - NPZ)],
                        z_out.at[cid, pl.ds(NPZ, NP - NPZ)])


def _sc_edge_body(u_hbm, v_hbm, rc_hbm, zeros_hbm, ec_out, degw_out,
                  u_v, v_v, IS, ecb, RW, deg_sh):
    cid = lax.axis_index("c")
    sid = lax.axis_index("s")
    wid = cid * 16 + sid
    r0 = sid * RPZ
    pltpu.sync_copy(u_hbm, u_v)
    pltpu.sync_copy(v_hbm, v_v)
    pltpu.sync_copy(zeros_hbm.at[pl.ds(r0, RPZ)], deg_sh.at[pl.ds(r0, RPZ)])
    plsc.subcore_barrier()

    def chunk(c, carry):
        s = c // SUP
        j = c - s * SUP

        @pl.when(j == 0)
        def _():
            pltpu.sync_copy(rc_hbm.at[wid, pl.ds(s * SUP, SUP)], IS)

        for g in range(K // 16):
            r16 = IS[j, 0, pl.ds(g * 16, 16)]
            c16 = IS[j, 1, pl.ds(g * 16, 16)]
            uu = plsc.load_gather(u_v, [r16])
            vv = plsc.load_gather(v_v, [c16])
            ec = 1.0 / (1.0 + jnp.exp(-(uu + vv)))
            ecb[pl.ds(g * 16, 16)] = ec

        def edge(e, cc):
            w16 = plsc.load_gather(ecb, [jnp.full((16,), e, i32)])
            w16o = 1.0 - w16
            for q in range(4):
                RW[e, pl.ds(q * 16, 16)] = w16
            for q in range(4, 8):
                RW[e, pl.ds(q * 16, 16)] = w16o
            del w16, w16o
            return cc

        lax.fori_loop(0, K, edge, 0)
        pltpu.sync_copy(ecb, ec_out.at[wid, c])
        pltpu.sync_copy(RW, deg_sh.at[IS.at[j, 0]], add=True)
        return carry

    lax.fori_loop(0, NCH, chunk, 0)
    plsc.subcore_barrier()
    pltpu.sync_copy(deg_sh.at[pl.ds(r0, RPZ)], degw_out.at[cid, pl.ds(r0, RPZ)])

    @pl.when(sid == 15)
    def _():
        pltpu.sync_copy(zeros_hbm.at[pl.ds(0, NP - NPZ)],
                        degw_out.at[cid, pl.ds(NPZ, NP - NPZ)])


def _sc_wspmm_body(y_hbm, rcb_hbm, ew_hbm, zeros_hbm, z_out, IS, EW, D, z_sh,
                   i0, i1, e0, e1, g0, g1, g2, g3, s0, s1, s2, s3):
    cid = lax.axis_index("c")
    sid = lax.axis_index("s")
    r0 = sid * RPZ
    isems = (i0, i1)
    esems = (e0, e1)
    gsems = (g0, g1, g2, g3)
    ssems = (s0, s1, s2, s3)
    fv = jnp.full((16,), cid.astype(f32), f32)
    a0 = fv
    a1 = 1.0 - 2.0 * fv

    def idx_dma(sb, s):
        pltpu.async_copy(rcb_hbm.at[cid, sid, pl.ds(s * SUPW, SUPW)],
                         IS.at[sb], isems[sb])
        pltpu.async_copy(ew_hbm.at[sid, pl.ds(s * SUPW, SUPW)],
                         EW.at[sb], esems[sb])

    def wait_i(sb):
        pltpu.make_async_copy(rcb_hbm.at[0, 0, pl.ds(0, SUPW)], IS.at[sb],
                              isems[sb]).wait()
        pltpu.make_async_copy(ew_hbm.at[0, pl.ds(0, SUPW)], EW.at[sb],
                              esems[sb]).wait()

    def gather(slot, s2, j):
        pltpu.async_copy(y_hbm.at[IS.at[s2, j, 0]], D.at[slot], gsems[slot])

    def wait_g(slot):
        pltpu.make_async_copy(y_hbm.at[IS.at[0, 0, 0]], D.at[slot],
                              gsems[slot]).wait()

    def scatter(b, s2, j):
        pltpu.async_copy(D.at[b], z_sh.at[IS.at[s2, j, 1]], ssems[b],
                         add=True)

    def wait_s(slot):
        pltpu.make_async_copy(D.at[slot], z_sh.at[IS.at[0, 0, 1]],
                              ssems[slot]).wait()

    idx_dma(0, 0)
    pltpu.sync_copy(zeros_hbm.at[pl.ds(r0, RPZ)], z_sh.at[pl.ds(r0, RPZ)])
    plsc.subcore_barrier()
    wait_i(0)
    gather(0, 0, 0)
    gather(1, 0, 1)

    def triple(t, carry):
        c0 = 3 * t
        for b in range(3):
            c = c0 + b
            s = c // SUPW
            j = c - s * SUPW
            s2 = s % 2
            if b >= 1:
                wait_s((b + 2) % 3)
            else:
                @pl.when(t > 0)
                def _():
                    wait_s(2)
            for kk in range(2):
                @pl.when((j == 0) & (s + 1 < NSUPW) & ((s + 1) % 2 == kk))
                def _():
                    idx_dma(kk, s + 1)
            for kk in range(2):
                @pl.when((j == SUPW - 2) & (s + 1 < NSUPW)
                         & ((s + 1) % 2 == kk))
                def _():
                    wait_i(kk)
            p = jnp.minimum(c + 2, NCHW - 1)
            sp = p // SUPW
            gather((b + 2) % 3, sp % 2, p - sp * SUPW)
            wait_g(b)
            sv = jnp.full((16,), s2, i32)
            jv = jnp.full((16,), j, i32)

            def rbody(r, cc):
                w16 = plsc.load_gather(EW, [sv, jv, jnp.full((16,), r, i32)])
                w16 = a0 + a1 * w16
                for fch in range(8):
                    sl = D[b, r, pl.ds(fch * 16, 16)]
                    D[b, r, pl.ds(fch * 16, 16)] = sl * w16
                return cc

            lax.fori_loop(0, K, rbody, 0)
            scatter(b, s2, j)
        return carry

    lax.fori_loop(0, NCHW // 3, triple, 0)
    wait_g(0)
    wait_g(1)
    wait_s(2)
    plsc.subcore_barrier()
    pltpu.sync_copy(z_sh.at[pl.ds(r0, RPZ)], z_out.at[cid, pl.ds(r0, RPZ)])

    @pl.when(sid == 15)
    def _():
        pltpu.sync_copy(zeros_hbm.at[pl.ds(0, NP - NPZ)],
                        z_out.at[cid, pl.ds(NPZ, NP - NPZ)])


def _mk_mesh():
    return plsc.VectorSubcoreMesh(core_axis_name="c", subcore_axis_name="s")


def _sc_hist(row_t, ones_kh, zeros_nph):
    k = functools.partial(
        pl.kernel,
        compiler_params=pltpu.CompilerParams(needs_layout_passes=False),
        out_type=jax.ShapeDtypeStruct((2, NP, H), f32),
        mesh=_mk_mesh(),
        scratch_types=[pltpu.VMEM((NCH, K), i32), pltpu.VMEM((K, H), f32),
                       pltpu.VMEM_SHARED((NP, H), f32)])(_sc_hist_body)
    return k(row_t, ones_kh, zeros_nph)


def _sc_spmm(y, rc_t, zeros_nph):
    k = functools.partial(
        pl.kernel,
        compiler_params=pltpu.CompilerParams(needs_layout_passes=False),
        out_type=jax.ShapeDtypeStruct((2, NP, H), f32),
        mesh=_mk_mesh(),
        scratch_types=[pltpu.VMEM((2, SUP, 2, K), i32),
                       pltpu.VMEM((ND, K, H), f32),
                       pltpu.VMEM_SHARED((NPZ, H), f32)]
        + [pltpu.SemaphoreType.DMA] * 10)(_sc_spmm_body)
    return k(y, rc_t, zeros_nph)


def _sc_edge(u, v, rc_t, zeros_nph):
    k = functools.partial(
        pl.kernel,
        compiler_params=pltpu.CompilerParams(needs_layout_passes=False),
        out_type=(jax.ShapeDtypeStruct((NT, NCH, K), f32),
                  jax.ShapeDtypeStruct((2, NP, H), f32)),
        mesh=_mk_mesh(),
        scratch_types=[pltpu.VMEM((NP,), f32), pltpu.VMEM((NP,), f32),
                       pltpu.VMEM((SUP, 2, K), i32), pltpu.VMEM((K,), f32),
                       pltpu.VMEM((K, H), f32),
                       pltpu.VMEM_SHARED((NPZ, H), f32)])(
        _sc_edge_body)
    return k(u, v, rc_t, zeros_nph)


def _sc_wspmm(y2, rcb, ew, zeros_nph):
    k = functools.partial(
        pl.kernel,
        compiler_params=pltpu.CompilerParams(needs_layout_passes=False),
        out_type=jax.ShapeDtypeStruct((2, NP, H), f32),
        mesh=_mk_mesh(),
        scratch_types=[pltpu.VMEM((2, SUPW, 2, K), i32),
                       pltpu.VMEM((2, SUPW, K), f32),
                       pltpu.VMEM((3, K, H), f32),
                       pltpu.VMEM_SHARED((NPZ, H), f32)]
        + [pltpu.SemaphoreType.DMA] * 12)(_sc_wspmm_body)
    return k(y2, rcb, ew, zeros_nph)


def kernel(x, W_feat, b_feat, W0, b0, W1, b1, W2, b2, We, be, Wn, bn_b,
           Wc, bc, Wo, bo, W1c, b1c, W2c, b2c, W1o, b1o, W2o, b2o,
           W1co, b1co, W2co, b2co, edge_index, batch):
    row = edge_index[0].astype(i32)
    col = edge_index[1].astype(i32)
    padn = jnp.full((NT * NCH * K - E,), N, i32)
    row_t = jnp.concatenate([row, padn]).reshape(NT, NCH, K)
    col_t = jnp.concatenate([col, padn]).reshape(NT, NCH, K)
    rc_t = jnp.stack([row_t, col_t], axis=2)
    xp = jnp.zeros((NP, x.shape[1]), f32).at[:N].set(x)
    zeros_nph = jnp.zeros((NP, H), f32)
    ones_kh = jnp.ones((K, H), f32)
    rowv = lambda a: a.reshape(1, H)
    bf = rowv(b_feat)
    We_p = jnp.zeros((2 * H, H), f32).at[:, :2].set(We)
    be_p = jnp.zeros((1, H), f32).at[0, :2].set(be)
    Wn_p = jnp.zeros((H, H), f32).at[:, :2].set(Wn)
    bnb_p = jnp.zeros((1, H), f32).at[0, :2].set(bn_b)
    bp = jnp.concatenate([batch.astype(i32), jnp.full((NP - N,), G, i32)])
    oh = (bp[:, None] == jnp.arange(G, dtype=i32)[None, :]).astype(f32)
    W2c_p = jnp.zeros((H, H), f32).at[:, :C].set(W2c)
    b2c_p = jnp.zeros((1, H), f32).at[0, :C].set(b2c)
    W2o_p = jnp.zeros((H, H), f32).at[:, :C].set(W2o)
    b2o_p = jnp.zeros((1, H), f32).at[0, :C].set(b2o)
    W2co_p = jnp.zeros((H, H), f32).at[:, :C].set(W2co)
    b2co_p = jnp.zeros((1, H), f32).at[0, :C].set(b2co)

    degu = _sc_hist(row_t, ones_kh, zeros_nph)
    sx = _tc_stats(xp)
    h, sh = _tc_feat(xp, sx, W_feat, bf)

    for (Wi, bi) in ((W0, b0), (W1, b1), (W2, b2)):
        xw, y = _tc_front(h, sh, degu, Wi)
        z = _sc_spmm(y, rc_t, zeros_nph)
        h, sh = _tc_back(z, xw, degu, rowv(bi))

    u, v, xc, xo, sxc, sxo = _tc_att(h, We_p, be_p, Wn_p, bnb_p)
    ec, degw = _sc_edge(u.reshape(NP), v.reshape(NP), rc_t, zeros_nph)

    xww, yw = _tc_wfront(xc, xo, sxc, sxo, degw, Wc, Wo)
    y2 = yw.reshape(2 * NP, H)
    NCHP = NSUPW * SUPW
    cpad = jnp.full((16, NCHP - NCHW, K), N, i32)
    row_w = jnp.concatenate([row_t.reshape(16, NCHW, K), cpad], axis=1)
    col_w = jnp.concatenate([col_t.reshape(16, NCHW, K), cpad], axis=1)
    rcb = jnp.stack([jnp.stack([row_w, col_w], axis=2),
                     jnp.stack([row_w + NP, col_w], axis=2)], axis=0)
    ew_w = jnp.concatenate(
        [ec.reshape(16, NCHW, K), jnp.zeros((16, NCHP - NCHW, K), f32)],
        axis=1)
    zw = _sc_wspmm(y2, rcb, ew_w, zeros_nph)

    xcg, xog = _tc_final(zw, xww, degw, rowv(bc), rowv(bo), oh)
    ws = (W1c, rowv(b1c), W2c_p, b2c_p, W1o, rowv(b1o), W2o_p, b2o_p,
          W1co[:H], W1co[H:], rowv(b1co), W2co_p, b2co_p)
    oc, oo, oco = _tc_readout(xcg, xog, ws)
    return (oc[:, :C], oo[:, :C], oco[:, :C])


# speedup vs baseline: 1.0387x; 1.0387x over previous
import functools

import jax
import jax.numpy as jnp
from jax import lax
from jax.experimental import pallas as pl
from jax.experimental.pallas import tpu as pltpu
from jax.experimental.pallas import tpu_sc as plsc

N = 10000
E = 320000
H = 128
C = 10
G = 128

NP = 10240
NPZ = 10112
K = 112
NCH = 90
NT = 32
NCHW = 2 * NCH
SUP = 6
SUPW = 8
NSUPW = 23
ND = 3
RB = 1024
NBLK = NP // RB
RPT = NP // 16
RPZ = NPZ // 16

f32 = jnp.float32
i32 = jnp.int32


def _rowmask(i):
    rows = lax.broadcasted_iota(i32, (RB, 1), 0) + i * RB
    return (rows < N).astype(f32)


def _affine(s_ref):
    s0 = s_ref[0:1, :]
    s1 = s_ref[1:2, :]
    m = s0 * (1.0 / N)
    v = s1 * (1.0 / N) - m * m
    al = lax.rsqrt(v + 1e-5)
    return al, -m * al + 1e-4


def _stats_of(x, i, s_ref):
    xm = x * _rowmask(i)
    part = jnp.concatenate(
        [jnp.sum(xm, 0, keepdims=True), jnp.sum(xm * xm, 0, keepdims=True),
         jnp.zeros((6, H), f32)], axis=0)

    @pl.when(i == 0)
    def _():
        s_ref[...] = part

    @pl.when(i > 0)
    def _():
        s_ref[...] = s_ref[...] + part


def _tc_stats_body(x_ref, s_ref):
    _stats_of(x_ref[...], pl.program_id(0), s_ref)


def _tc_feat_body(x_ref, sx_ref, Wf_ref, bf_ref, h_ref, s1_ref):
    i = pl.program_id(0)
    al, bt = _affine(sx_ref)
    xn = x_ref[...] * al + bt
    h = jnp.maximum(jnp.dot(xn, Wf_ref[...], preferred_element_type=f32)
                    + bf_ref[...], 0.0)
    h_ref[...] = h
    _stats_of(h, i, s1_ref)


def _tc_front_body(h_ref, sh_ref, deg_ref, W_ref, xw_ref, y_ref):
    i = pl.program_id(0)
    al, bt = _affine(sh_ref)
    xn = h_ref[...] * al + bt
    xw = jnp.dot(xn, W_ref[...], preferred_element_type=f32)
    dis = lax.rsqrt(deg_ref[0] + deg_ref[1] + 1.0)
    xw_ref[...] = xw
    y_ref[...] = dis * xw * _rowmask(i)


def _tc_back_body(z_ref, xw_ref, deg_ref, b_ref, h_ref, s_ref):
    i = pl.program_id(0)
    dis = lax.rsqrt(deg_ref[0] + deg_ref[1] + 1.0)
    xw = xw_ref[...]
    h = jnp.maximum(dis * (z_ref[0] + z_ref[1]) + dis * dis * xw + b_ref[...],
                    0.0)
    h_ref[...] = h
    _stats_of(h, i, s_ref)


def _tc_att_body(h_ref, We_ref, be_ref, Wn_ref, bnb_ref,
                 u_ref, v_ref, xc_ref, xo_ref, sc_ref, so_ref):
    i = pl.program_id(0)
    hb = h_ref[...]
    lanes = lax.broadcasted_iota(i32, (1, H), 1)
    sel = jnp.where(lanes == 0, 1.0, jnp.where(lanes == 1, -1.0, 0.0))
    wuv = lax.dot_general(sel, We_ref[...], (((1,), (1,)), ((), ())),
                          preferred_element_type=f32)
    wu = wuv[:, 0:H]
    wv = wuv[:, H:2 * H]
    be_d = jnp.sum(be_ref[...] * sel, axis=1, keepdims=True)
    urows = []
    vrows = []
    for sb in range(RB // 128):
        hs = hb[sb * 128:(sb + 1) * 128, :]
        urows.append(lax.dot_general(wu, hs, (((1,), (1,)), ((), ())),
                                     preferred_element_type=f32) + be_d)
        vrows.append(lax.dot_general(wv, hs, (((1,), (1,)), ((), ())),
                                     preferred_element_type=f32))
    u_ref[...] = jnp.concatenate(urows, axis=0)
    v_ref[...] = jnp.concatenate(vrows, axis=0)
    nl = jnp.dot(hb, Wn_ref[...], preferred_element_type=f32) + bnb_ref[...]
    d01 = nl[:, 0:1] - nl[:, 1:2]
    att0 = 1.0 / (1.0 + jnp.exp(-d01))
    xc = att0 * hb
    xo = hb - xc
    xc_ref[...] = xc
    xo_ref[...] = xo
    _stats_of(xc, i, sc_ref)
    _stats_of(xo, i, so_ref)


def _split_dis(dgw_ref):
    dsum = dgw_ref[0] + dgw_ref[1]
    ri = lax.broadcasted_iota(i32, (H, H), 0)
    s0 = (ri == 0).astype(f32)
    s64 = (ri == 64).astype(f32)
    degc = jnp.dot(dsum, s0, preferred_element_type=f32)
    dego = jnp.dot(dsum, s64, preferred_element_type=f32)
    return lax.rsqrt(degc + 1.0), lax.rsqrt(dego + 1.0)


def _tc_wfront_body(xc_ref, xo_ref, sc_ref, so_ref, dgw_ref,
                    Wc_ref, Wo_ref, xw_ref, y_ref):
    i = pl.program_id(0)
    msk = _rowmask(i)
    disc, diso = _split_dis(dgw_ref)
    alc, btc = _affine(sc_ref)
    xwc = jnp.dot(xc_ref[...] * alc + btc, Wc_ref[...],
                  preferred_element_type=f32)
    alo, bto = _affine(so_ref)
    xwo = jnp.dot(xo_ref[...] * alo + bto, Wo_ref[...],
                  preferred_element_type=f32)
    xw_ref[0] = xwc
    xw_ref[1] = xwo
    y_ref[0] = disc * xwc * msk
    y_ref[1] = diso * xwo * msk


def _tc_final_body(zw_ref, xw_ref, dgw_ref, bc_ref, bo_ref, oh_ref,
                   pc_ref, po_ref):
    i = pl.program_id(0)
    disc, diso = _split_dis(dgw_ref)
    xc2 = jnp.maximum(disc * zw_ref[0] + disc * disc * xw_ref[0] + bc_ref[...],
                      0.0)
    xo2 = jnp.maximum(diso * zw_ref[1] + diso * diso * xw_ref[1] + bo_ref[...],
                      0.0)
    oh = oh_ref[...]
    pc = lax.dot_general(oh, xc2, (((0,), (0,)), ((), ())),
                         preferred_element_type=f32)
    po = lax.dot_general(oh, xo2, (((0,), (0,)), ((), ())),
                         preferred_element_type=f32)

    @pl.when(i == 0)
    def _():
        pc_ref[...] = pc
        po_ref[...] = po

    @pl.when(i > 0)
    def _():
        pc_ref[...] = pc_ref[...] + pc
        po_ref[...] = po_ref[...] + po


def _gstats(x):
    s0 = jnp.sum(x, 0, keepdims=True)
    s1 = jnp.sum(x * x, 0, keepdims=True)
    m = s0 * (1.0 / G)
    v = s1 * (1.0 / G) - m * m
    al = lax.rsqrt(v + 1e-5)
    return al, -m * al + 1e-4


def _logsm(lo):
    lanes = lax.broadcasted_iota(i32, (G, H), 1)
    lom = jnp.where(lanes < C, lo, -1e30)
    mx = jnp.max(lom, axis=1, keepdims=True)
    ls = jnp.log(jnp.sum(jnp.exp(lom - mx), axis=1, keepdims=True))
    return lo - mx - ls


def _tc_readout_body(xcg_ref, xog_ref, W1c_ref, b1c_ref, W2c_ref, b2c_ref,
                     W1o_ref, b1o_ref, W2o_ref, b2o_ref,
                     W1t_ref, W1b_ref, b1co_ref, W2co_ref, b2co_ref,
                     oc_ref, oo_ref, oco_ref):
    xcg = xcg_ref[...]
    xog = xog_ref[...]

    def head(xg, W1r, b1r, W2r, b2r):
        al, bt = _gstats(xg)
        hh = jnp.maximum(jnp.dot(xg * al + bt, W1r,
                                 preferred_element_type=f32) + b1r, 0.0)
        al2, bt2 = _gstats(hh)
        return _logsm(jnp.dot(hh * al2 + bt2, W2r,
                              preferred_element_type=f32) + b2r)

    oc_ref[...] = head(xcg, W1c_ref[...], b1c_ref[...], W2c_ref[...],
                       b2c_ref[...])
    oo_ref[...] = head(xog, W1o_ref[...], b1o_ref[...], W2o_ref[...],
                       b2o_ref[...])
    alc, btc = _gstats(xcg)
    alo, bto = _gstats(xog)
    hh = jnp.maximum(
        jnp.dot(xcg * alc + btc, W1t_ref[...], preferred_element_type=f32)
        + jnp.dot(xog * alo + bto, W1b_ref[...], preferred_element_type=f32)
        + b1co_ref[...], 0.0)
    al2, bt2 = _gstats(hh)
    oco_ref[...] = _logsm(jnp.dot(hh * al2 + bt2, W2co_ref[...],
                                  preferred_element_type=f32) + b2co_ref[...])


_B_NH = pl.BlockSpec((RB, H), lambda i: (i, 0))
_B_2NH = pl.BlockSpec((2, RB, H), lambda i: (0, i, 0))
_B_S = pl.BlockSpec((8, H), lambda i: (0, 0))
_B_W = pl.BlockSpec((H, H), lambda i: (0, 0))
_B_B = pl.BlockSpec((1, H), lambda i: (0, 0))
_B_U = pl.BlockSpec((RB // 128, H), lambda i: (i, 0))
_B_G = pl.BlockSpec((G, H), lambda i: (0, 0))
_SNH = jax.ShapeDtypeStruct((NP, H), f32)
_S2NH = jax.ShapeDtypeStruct((2, NP, H), f32)
_SS = jax.ShapeDtypeStruct((8, H), f32)
_SU = jax.ShapeDtypeStruct((NP // H, H), f32)
_SG = jax.ShapeDtypeStruct((G, H), f32)


def _tc_stats(x):
    return pl.pallas_call(_tc_stats_body, grid=(NBLK,), in_specs=[_B_NH],
                          out_specs=_B_S, out_shape=_SS)(x)


def _tc_feat(x, sx, Wf, bf):
    return pl.pallas_call(
        _tc_feat_body, grid=(NBLK,),
        in_specs=[_B_NH, _B_S, _B_W, _B_B],
        out_specs=[_B_NH, _B_S], out_shape=[_SNH, _SS])(x, sx, Wf, bf)


def _tc_front(h, sh, degb, W):
    return pl.pallas_call(
        _tc_front_body, grid=(NBLK,),
        in_specs=[_B_NH, _B_S, _B_2NH, _B_W],
        out_specs=[_B_NH, _B_NH], out_shape=[_SNH, _SNH])(h, sh, degb, W)


def _tc_back(z, xw, degb, b):
    return pl.pallas_call(
        _tc_back_body, grid=(NBLK,),
        in_specs=[_B_2NH, _B_NH, _B_2NH, _B_B],
        out_specs=[_B_NH, _B_S], out_shape=[_SNH, _SS])(z, xw, degb, b)


def _tc_att(h, We_p, be_p, Wn_p, bnb_p):
    return pl.pallas_call(
        _tc_att_body, grid=(NBLK,),
        in_specs=[_B_NH, pl.BlockSpec((2 * H, H), lambda i: (0, 0)), _B_B,
                  _B_W, _B_B],
        out_specs=[_B_U, _B_U, _B_NH, _B_NH, _B_S, _B_S],
        out_shape=[_SU, _SU, _SNH, _SNH, _SS, _SS])(h, We_p, be_p, Wn_p, bnb_p)


def _tc_wfront(xc, xo, sc, so, dgw, Wc, Wo):
    return pl.pallas_call(
        _tc_wfront_body, grid=(NBLK,),
        in_specs=[_B_NH, _B_NH, _B_S, _B_S, _B_2NH, _B_W, _B_W],
        out_specs=[_B_2NH, _B_2NH],
        out_shape=[_S2NH, _S2NH])(xc, xo, sc, so, dgw, Wc, Wo)


def _tc_final(zw, xw, dgw, bc, bo, oh):
    return pl.pallas_call(
        _tc_final_body, grid=(NBLK,),
        in_specs=[_B_2NH, _B_2NH, _B_2NH, _B_B, _B_B,
                  pl.BlockSpec((RB, G), lambda i: (i, 0))],
        out_specs=[_B_G, _B_G], out_shape=[_SG, _SG])(zw, xw, dgw, bc, bo, oh)


def _tc_readout(xcg, xog, ws):
    gspec = pl.BlockSpec((G, H), lambda: (0, 0))
    bspec = pl.BlockSpec((1, H), lambda: (0, 0))
    specs = [gspec, gspec] + [gspec if w.shape[0] == H else bspec for w in ws]
    return pl.pallas_call(
        _tc_readout_body, grid=(),
        in_specs=specs, out_specs=[gspec, gspec, gspec],
        out_shape=[_SG, _SG, _SG])(xcg, xog, *ws)


def _sc_hist_body(row_hbm, ones_hbm, zeros_hbm, deg_out,
                  row_v, ones_v, deg_sh):
    cid = lax.axis_index("c")
    sid = lax.axis_index("s")
    wid = cid * 16 + sid
    r0 = sid * RPT
    pltpu.sync_copy(row_hbm.at[wid], row_v)
    pltpu.sync_copy(ones_hbm, ones_v)
    pltpu.sync_copy(zeros_hbm.at[pl.ds(r0, RPT)], deg_sh.at[pl.ds(r0, RPT)])
    plsc.subcore_barrier()

    def step(jb, c):
        pltpu.sync_copy(ones_v, deg_sh.at[row_v.at[jb]], add=True)
        return c

    lax.fori_loop(0, NCH, step, 0)
    plsc.subcore_barrier()
    pltpu.sync_copy(deg_sh.at[pl.ds(r0, RPT)], deg_out.at[cid, pl.ds(r0, RPT)])


def _sc_spmm_body(y_hbm, rc_hbm, zeros_hbm, z_out, IS, D, z_sh,
                  i0, i1, g0, g1, g2, g3, s0, s1, s2, s3):
    cid = lax.axis_index("c")
    sid = lax.axis_index("s")
    wid = cid * 16 + sid
    r0 = sid * RPZ
    isems = (i0, i1)
    gsems = (g0, g1, g2, g3)
    ssems = (s0, s1, s2, s3)
    NSUP = NCH // SUP

    def idx_dma(s):
        pltpu.async_copy(rc_hbm.at[wid, pl.ds(s * SUP, SUP)], IS.at[s % 2],
                         isems[s % 2])

    def wait_i(s):
        pltpu.make_async_copy(rc_hbm.at[0, pl.ds(0, SUP)], IS.at[s % 2],
                              isems[s % 2]).wait()

    def gather(c, slot):
        s, j = divmod(c, SUP)
        pltpu.async_copy(y_hbm.at[IS.at[s % 2, j, 0]], D.at[slot],
                         gsems[slot])

    def wait_g(slot):
        pltpu.make_async_copy(y_hbm.at[IS.at[0, 0, 0]], D.at[slot],
                              gsems[slot]).wait()

    def scatter(c):
        s, j = divmod(c, SUP)
        pltpu.async_copy(D.at[c % ND], z_sh.at[IS.at[s % 2, j, 1]],
                         ssems[c % ND], add=True)

    def wait_s(slot):
        pltpu.make_async_copy(D.at[slot], z_sh.at[IS.at[0, 0, 1]],
                              ssems[slot]).wait()

    idx_dma(0)
    pltpu.sync_copy(zeros_hbm.at[pl.ds(r0, RPZ)], z_sh.at[pl.ds(r0, RPZ)])
    plsc.subcore_barrier()
    wait_i(0)
    gather(0, 0)
    gather(1, 1)
    swaited = set()

    def scatter_done(cc):
        if cc >= 0 and cc not in swaited:
            wait_s(cc % ND)
            swaited.add(cc)

    for c in range(NCH):
        s, j = divmod(c, SUP)
        if j == 0 and s + 1 < NSUP:
            scatter_done(c - 2)
            scatter_done(c - 1)
            idx_dma(s + 1)
        if j == SUP - 2 and s + 1 < NSUP:
            wait_i(s + 1)
        scatter_done(c + 2 - ND)
        gather(min(c + 2, NCH - 1), (c + 2) % ND)
        wait_g(c % ND)
        scatter(c)
    wait_g((NCH) % ND)
    wait_g((NCH + 1) % ND)
    scatter_done(NCH - 2)
    scatter_done(NCH - 1)
    plsc.subcore_barrier()
    pltpu.sync_copy(z_sh.at[pl.ds(r0, RPZ)], z_out.at[cid, pl.ds(r0, RPZ)])

    @pl.when(sid == 15)
    def _():
        pltpu.sync_copy(zeros_hbm.at[pl.ds(0, NP - NPZ)],
                        z_out.at[cid, pl.ds(NPZ, NP - NPZ)])


def _sc_edge_body(u_hbm, v_hbm, rc_hbm, zeros_hbm, ec_out, degw_out,
                  u_v, v_v, IS, ecb, RW, deg_sh):
    cid = lax.axis_index("c")
    sid = lax.axis_index("s")
    wid = cid * 16 + sid
    r0 = sid * RPZ
    pltpu.sync_copy(u_hbm, u_v)
    pltpu.sync_copy(v_hbm, v_v)
    pltpu.sync_copy(zeros_hbm.at[pl.ds(r0, RPZ)], deg_sh.at[pl.ds(r0, RPZ)])
    plsc.subcore_barrier()

    def chunk(c, carry):
        s = c // SUP
        j = c - s * SUP

        @pl.when(j == 0)
        def _():
            pltpu.sync_copy(rc_hbm.at[wid, pl.ds(s * SUP, SUP)], IS)

        for g in range(K // 16):
            r16 = IS[j, 0, pl.ds(g * 16, 16)]
            c16 = IS[j, 1, pl.ds(g * 16, 16)]
            uu = plsc.load_gather(u_v, [r16])
            vv = plsc.load_gather(v_v, [c16])
            ec = 1.0 / (1.0 + jnp.exp(-(uu + vv)))
            ecb[pl.ds(g * 16, 16)] = ec

        def edge(e, cc):
            w16 = plsc.load_gather(ecb, [jnp.full((16,), e, i32)])
            w16o = 1.0 - w16
            for q in range(4):
                RW[e, pl.ds(q * 16, 16)] = w16
            for q in range(4, 8):
                RW[e, pl.ds(q * 16, 16)] = w16o
            del w16, w16o
            return cc

        lax.fori_loop(0, K, edge, 0)
        pltpu.sync_copy(ecb, ec_out.at[wid, c])
        pltpu.sync_copy(RW, deg_sh.at[IS.at[j, 0]], add=True)
        return carry

    lax.fori_loop(0, NCH, chunk, 0)
    plsc.subcore_barrier()
    pltpu.sync_copy(deg_sh.at[pl.ds(r0, RPZ)], degw_out.at[cid, pl.ds(r0, RPZ)])

    @pl.when(sid == 15)
    def _():
        pltpu.sync_copy(zeros_hbm.at[pl.ds(0, NP - NPZ)],
                        degw_out.at[cid, pl.ds(NPZ, NP - NPZ)])


def _sc_wspmm_body(y_hbm, rcb_hbm, ew_hbm, zeros_hbm, z_out, IS, EW, D, z_sh,
                   i0, i1, e0, e1, g0, g1, g2, g3, s0, s1, s2, s3):
    cid = lax.axis_index("c")
    sid = lax.axis_index("s")
    r0 = sid * RPZ
    isems = (i0, i1)
    esems = (e0, e1)
    gsems = (g0, g1, g2, g3)
    ssems = (s0, s1, s2, s3)
    fv = jnp.full((16,), cid.astype(f32), f32)
    a0 = fv
    a1 = 1.0 - 2.0 * fv

    def idx_dma(sb, s):
        pltpu.async_copy(rcb_hbm.at[cid, sid, pl.ds(s * SUPW, SUPW)],
                         IS.at[sb], isems[sb])
        pltpu.async_copy(ew_hbm.at[sid, pl.ds(s * SUPW, SUPW)],
                         EW.at[sb], esems[sb])

    def wait_i(sb):
        pltpu.make_async_copy(rcb_hbm.at[0, 0, pl.ds(0, SUPW)], IS.at[sb],
                              isems[sb]).wait()
        pltpu.make_async_copy(ew_hbm.at[0, pl.ds(0, SUPW)], EW.at[sb],
                              esems[sb]).wait()

    def gather(slot, s2, j):
        pltpu.async_copy(y_hbm.at[IS.at[s2, j, 0]], D.at[slot], gsems[slot])

    def wait_g(slot):
        pltpu.make_async_copy(y_hbm.at[IS.at[0, 0, 0]], D.at[slot],
                              gsems[slot]).wait()

    def scatter(b, s2, j):
        pltpu.async_copy(D.at[b], z_sh.at[IS.at[s2, j, 1]], ssems[b],
                         add=True)

    def wait_s(slot):
        pltpu.make_async_copy(D.at[slot], z_sh.at[IS.at[0, 0, 1]],
                              ssems[slot]).wait()

    idx_dma(0, 0)
    pltpu.sync_copy(zeros_hbm.at[pl.ds(r0, RPZ)], z_sh.at[pl.ds(r0, RPZ)])
    plsc.subcore_barrier()
    wait_i(0)
    gather(0, 0, 0)
    gather(1, 0, 1)

    def triple(t, carry):
        c0 = 3 * t
        for b in range(3):
            c = c0 + b
            s = c // SUPW
            j = c - s * SUPW
            s2 = s % 2
            if b >= 1:
                wait_s((b + 2) % 3)
            else:
                @pl.when(t > 0)
                def _():
                    wait_s(2)
            for kk in range(2):
                @pl.when((j == 0) & (s + 1 < NSUPW) & ((s + 1) % 2 == kk))
                def _():
                    idx_dma(kk, s + 1)
            for kk in range(2):
                @pl.when((j == SUPW - 2) & (s + 1 < NSUPW)
                         & ((s + 1) % 2 == kk))
                def _():
                    wait_i(kk)
            p = jnp.minimum(c + 2, NCHW - 1)
            sp = p // SUPW
            gather((b + 2) % 3, sp % 2, p - sp * SUPW)
            wait_g(b)
            sv = jnp.full((16,), s2, i32)
            jv = jnp.full((16,), j, i32)

            def rbody(r, cc):
                w16 = plsc.load_gather(EW, [sv, jv, jnp.full((16,), r, i32)])
                w16 = a0 + a1 * w16
                for fch in range(8):
                    sl = D[b, r, pl.ds(fch * 16, 16)]
                    D[b, r, pl.ds(fch * 16, 16)] = sl * w16
                return cc

            lax.fori_loop(0, K, rbody, 0)
            scatter(b, s2, j)
        return carry

    lax.fori_loop(0, NCHW // 3, triple, 0)
    wait_g(0)
    wait_g(1)
    wait_s(2)
    plsc.subcore_barrier()
    pltpu.sync_copy(z_sh.at[pl.ds(r0, RPZ)], z_out.at[cid, pl.ds(r0, RPZ)])

    @pl.when(sid == 15)
    def _():
        pltpu.sync_copy(zeros_hbm.at[pl.ds(0, NP - NPZ)],
                        z_out.at[cid, pl.ds(NPZ, NP - NPZ)])


def _mk_mesh():
    return plsc.VectorSubcoreMesh(core_axis_name="c", subcore_axis_name="s")


def _sc_hist(row_t, ones_kh, zeros_nph):
    k = functools.partial(
        pl.kernel,
        compiler_params=pltpu.CompilerParams(needs_layout_passes=False),
        out_type=jax.ShapeDtypeStruct((2, NP, H), f32),
        mesh=_mk_mesh(),
        scratch_types=[pltpu.VMEM((NCH, K), i32), pltpu.VMEM((K, H), f32),
                       pltpu.VMEM_SHARED((NP, H), f32)])(_sc_hist_body)
    return k(row_t, ones_kh, zeros_nph)


def _sc_spmm(y, rc_t, zeros_nph):
    k = functools.partial(
        pl.kernel,
        compiler_params=pltpu.CompilerParams(needs_layout_passes=False),
        out_type=jax.ShapeDtypeStruct((2, NP, H), f32),
        mesh=_mk_mesh(),
        scratch_types=[pltpu.VMEM((2, SUP, 2, K), i32),
                       pltpu.VMEM((ND, K, H), f32),
                       pltpu.VMEM_SHARED((NPZ, H), f32)]
        + [pltpu.SemaphoreType.DMA] * 10)(_sc_spmm_body)
    return k(y, rc_t, zeros_nph)


def _sc_edge(u, v, rc_t, zeros_nph):
    k = functools.partial(
        pl.kernel,
        compiler_params=pltpu.CompilerParams(needs_layout_passes=False),
        out_type=(jax.ShapeDtypeStruct((NT, NCH, K), f32),
                  jax.ShapeDtypeStruct((2, NP, H), f32)),
        mesh=_mk_mesh(),
        scratch_types=[pltpu.VMEM((NP,), f32), pltpu.VMEM((NP,), f32),
                       pltpu.VMEM((SUP, 2, K), i32), pltpu.VMEM((K,), f32),
                       pltpu.VMEM((K, H), f32),
                       pltpu.VMEM_SHARED((NPZ, H), f32)])(
        _sc_edge_body)
    return k(u, v, rc_t, zeros_nph)


def _sc_wspmm(y2, rcb, ew, zeros_nph):
    k = functools.partial(
        pl.kernel,
        compiler_params=pltpu.CompilerParams(needs_layout_passes=False),
        out_type=jax.ShapeDtypeStruct((2, NP, H), f32),
        mesh=_mk_mesh(),
        scratch_types=[pltpu.VMEM((2, SUPW, 2, K), i32),
                       pltpu.VMEM((2, SUPW, K), f32),
                       pltpu.VMEM((3, K, H), f32),
                       pltpu.VMEM_SHARED((NPZ, H), f32)]
        + [pltpu.SemaphoreType.DMA] * 12)(_sc_wspmm_body)
    return k(y2, rcb, ew, zeros_nph)


def kernel(x, W_feat, b_feat, W0, b0, W1, b1, W2, b2, We, be, Wn, bn_b,
           Wc, bc, Wo, bo, W1c, b1c, W2c, b2c, W1o, b1o, W2o, b2o,
           W1co, b1co, W2co, b2co, edge_index, batch):
    row = edge_index[0].astype(i32)
    col = edge_index[1].astype(i32)
    padn = jnp.full((NT * NCH * K - E,), N, i32)
    row_t = jnp.concatenate([row, padn]).reshape(NT, NCH, K)
    col_t = jnp.concatenate([col, padn]).reshape(NT, NCH, K)
    rc_t = jnp.stack([row_t, col_t], axis=2)
    xp = jnp.zeros((NP, x.shape[1]), f32).at[:N].set(x)
    zeros_nph = jnp.zeros((NP, H), f32)
    ones_kh = jnp.ones((K, H), f32)
    rowv = lambda a: a.reshape(1, H)
    bf = rowv(b_feat)
    We_p = jnp.zeros((2 * H, H), f32).at[:, :2].set(We)
    be_p = jnp.zeros((1, H), f32).at[0, :2].set(be)
    Wn_p = jnp.zeros((H, H), f32).at[:, :2].set(Wn)
    bnb_p = jnp.zeros((1, H), f32).at[0, :2].set(bn_b)
    bp = jnp.concatenate([batch.astype(i32), jnp.full((NP - N,), G, i32)])
    oh = (bp[:, None] == jnp.arange(G, dtype=i32)[None, :]).astype(f32)
    W2c_p = jnp.zeros((H, H), f32).at[:, :C].set(W2c)
    b2c_p = jnp.zeros((1, H), f32).at[0, :C].set(b2c)
    W2o_p = jnp.zeros((H, H), f32).at[:, :C].set(W2o)
    b2o_p = jnp.zeros((1, H), f32).at[0, :C].set(b2o)
    W2co_p = jnp.zeros((H, H), f32).at[:, :C].set(W2co)
    b2co_p = jnp.zeros((1, H), f32).at[0, :C].set(b2co)

    degu = _sc_hist(row_t, ones_kh, zeros_nph)
    sx = _tc_stats(xp)
    h, sh = _tc_feat(xp, sx, W_feat, bf)

    for (Wi, bi) in ((W0, b0), (W1, b1), (W2, b2)):
        xw, y = _tc_front(h, sh, degu, Wi)
        z = _sc_spmm(y, rc_t, zeros_nph)
        h, sh = _tc_back(z, xw, degu, rowv(bi))

    u, v, xc, xo, sxc, sxo = _tc_att(h, We_p, be_p, Wn_p, bnb_p)
    ec, degw = _sc_edge(u.reshape(NP), v.reshape(NP), rc_t, zeros_nph)

    xww, yw = _tc_wfront(xc, xo, sxc, sxo, degw, Wc, Wo)
    y2 = yw.reshape(2 * NP, H)
    NCHP = NSUPW * SUPW
    cpad = jnp.full((16, NCHP - NCHW, K), N, i32)
    row_w = jnp.concatenate([row_t.reshape(16, NCHW, K), cpad], axis=1)
    col_w = jnp.concatenate([col_t.reshape(16, NCHW, K), cpad], axis=1)
    rcb = jnp.stack([jnp.stack([row_w, col_w], axis=2),
                     jnp.stack([row_w + NP, col_w], axis=2)], axis=0)
    ew_w = jnp.concatenate(
        [ec.reshape(16, NCHW, K), jnp.zeros((16, NCHP - NCHW, K), f32)],
        axis=1)
    zw = _sc_wspmm(y2, rcb, ew_w, zeros_nph)

    xcg, xog = _tc_final(zw, xww, degw, rowv(bc), rowv(bo), oh)
    ws = (W1c, rowv(b1c), W2c_p, b2c_p, W1o, rowv(b1o), W2o_p, b2o_p,
          W1co[:H], W1co[H:], rowv(b1co), W2co_p, b2co_p)
    oc, oo, oco = _tc_readout(xcg, xog, ws)
    return (oc[:, :C], oo[:, :C], oco[:, :C])


# speedup vs baseline: 1.0856x; 1.0452x over previous
import functools

import jax
import jax.numpy as jnp
from jax import lax
from jax.experimental import pallas as pl
from jax.experimental.pallas import tpu as pltpu
from jax.experimental.pallas import tpu_sc as plsc

N = 10000
E = 320000
H = 128
C = 10
G = 128

NP = 10240
NPZ = 10112
K = 112
NCH = 90
NT = 32
NCHW = 2 * NCH
SUP = 6
SUPW = 8
NSUPW = 23
ND = 3
NCH0 = 120
NCH1 = 2 * NCH - NCH0
RB = 1024
NBLK = NP // RB
RPT = NP // 16
RPZ = NPZ // 16

f32 = jnp.float32
i32 = jnp.int32


def _rowmask(i):
    rows = lax.broadcasted_iota(i32, (RB, 1), 0) + i * RB
    return (rows < N).astype(f32)


def _affine(s_ref):
    s0 = s_ref[0:1, :]
    s1 = s_ref[1:2, :]
    m = s0 * (1.0 / N)
    v = s1 * (1.0 / N) - m * m
    al = lax.rsqrt(v + 1e-5)
    return al, -m * al + 1e-4


def _stats_of(x, i, s_ref):
    xm = x * _rowmask(i)
    part = jnp.concatenate(
        [jnp.sum(xm, 0, keepdims=True), jnp.sum(xm * xm, 0, keepdims=True),
         jnp.zeros((6, H), f32)], axis=0)

    @pl.when(i == 0)
    def _():
        s_ref[...] = part

    @pl.when(i > 0)
    def _():
        s_ref[...] = s_ref[...] + part


def _tc_stats_body(x_ref, s_ref):
    _stats_of(x_ref[...], pl.program_id(0), s_ref)


def _tc_feat_body(x_ref, sx_ref, Wf_ref, bf_ref, h_ref, s1_ref):
    i = pl.program_id(0)
    al, bt = _affine(sx_ref)
    xn = x_ref[...] * al + bt
    h = jnp.maximum(jnp.dot(xn, Wf_ref[...], preferred_element_type=f32)
                    + bf_ref[...], 0.0)
    h_ref[...] = h
    _stats_of(h, i, s1_ref)


def _tc_front_body(h_ref, sh_ref, deg_ref, W_ref, xw_ref, y_ref):
    i = pl.program_id(0)
    al, bt = _affine(sh_ref)
    xn = h_ref[...] * al + bt
    xw = jnp.dot(xn, W_ref[...], preferred_element_type=f32)
    dis = lax.rsqrt(deg_ref[0] + deg_ref[1] + 1.0)
    xw_ref[...] = xw
    y_ref[...] = dis * xw * _rowmask(i)


def _tc_back_body(z_ref, xw_ref, deg_ref, b_ref, h_ref, s_ref):
    i = pl.program_id(0)
    dis = lax.rsqrt(deg_ref[0] + deg_ref[1] + 1.0)
    xw = xw_ref[...]
    h = jnp.maximum(dis * (z_ref[0] + z_ref[1]) + dis * dis * xw + b_ref[...],
                    0.0)
    h_ref[...] = h
    _stats_of(h, i, s_ref)


def _tc_att_body(h_ref, We_ref, be_ref, Wn_ref, bnb_ref,
                 u_ref, v_ref, xc_ref, xo_ref, sc_ref, so_ref):
    i = pl.program_id(0)
    hb = h_ref[...]
    lanes = lax.broadcasted_iota(i32, (1, H), 1)
    sel = jnp.where(lanes == 0, 1.0, jnp.where(lanes == 1, -1.0, 0.0))
    wuv = lax.dot_general(sel, We_ref[...], (((1,), (1,)), ((), ())),
                          preferred_element_type=f32)
    wu = wuv[:, 0:H]
    wv = wuv[:, H:2 * H]
    be_d = jnp.sum(be_ref[...] * sel, axis=1, keepdims=True)
    urows = []
    vrows = []
    for sb in range(RB // 128):
        hs = hb[sb * 128:(sb + 1) * 128, :]
        urows.append(lax.dot_general(wu, hs, (((1,), (1,)), ((), ())),
                                     preferred_element_type=f32) + be_d)
        vrows.append(lax.dot_general(wv, hs, (((1,), (1,)), ((), ())),
                                     preferred_element_type=f32))
    u_ref[...] = jnp.concatenate(urows, axis=0)
    v_ref[...] = jnp.concatenate(vrows, axis=0)
    nl = jnp.dot(hb, Wn_ref[...], preferred_element_type=f32) + bnb_ref[...]
    d01 = nl[:, 0:1] - nl[:, 1:2]
    att0 = 1.0 / (1.0 + jnp.exp(-d01))
    xc = att0 * hb
    xo = hb - xc
    xc_ref[...] = xc
    xo_ref[...] = xo
    _stats_of(xc, i, sc_ref)
    _stats_of(xo, i, so_ref)


def _split_dis(dgw_ref):
    dsum = dgw_ref[0] + dgw_ref[1]
    ri = lax.broadcasted_iota(i32, (H, H), 0)
    s0 = (ri == 0).astype(f32)
    s64 = (ri == 64).astype(f32)
    degc = jnp.dot(dsum, s0, preferred_element_type=f32)
    dego = jnp.dot(dsum, s64, preferred_element_type=f32)
    return lax.rsqrt(degc + 1.0), lax.rsqrt(dego + 1.0)


def _tc_wfront_body(xc_ref, xo_ref, sc_ref, so_ref, dgw_ref,
                    Wc_ref, Wo_ref, xw_ref, y_ref):
    i = pl.program_id(0)
    msk = _rowmask(i)
    disc, diso = _split_dis(dgw_ref)
    alc, btc = _affine(sc_ref)
    xwc = jnp.dot(xc_ref[...] * alc + btc, Wc_ref[...],
                  preferred_element_type=f32)
    alo, bto = _affine(so_ref)
    xwo = jnp.dot(xo_ref[...] * alo + bto, Wo_ref[...],
                  preferred_element_type=f32)
    xw_ref[0] = xwc
    xw_ref[1] = xwo
    y_ref[0] = disc * xwc * msk
    y_ref[1] = diso * xwo * msk


def _tc_final_body(zw_ref, xw_ref, dgw_ref, bc_ref, bo_ref, oh_ref,
                   pc_ref, po_ref):
    i = pl.program_id(0)
    disc, diso = _split_dis(dgw_ref)
    xc2 = jnp.maximum(disc * zw_ref[0] + disc * disc * xw_ref[0] + bc_ref[...],
                      0.0)
    xo2 = jnp.maximum(diso * zw_ref[1] + diso * diso * xw_ref[1] + bo_ref[...],
                      0.0)
    oh = oh_ref[...]
    pc = lax.dot_general(oh, xc2, (((0,), (0,)), ((), ())),
                         preferred_element_type=f32)
    po = lax.dot_general(oh, xo2, (((0,), (0,)), ((), ())),
                         preferred_element_type=f32)

    @pl.when(i == 0)
    def _():
        pc_ref[...] = pc
        po_ref[...] = po

    @pl.when(i > 0)
    def _():
        pc_ref[...] = pc_ref[...] + pc
        po_ref[...] = po_ref[...] + po


def _gstats(x):
    s0 = jnp.sum(x, 0, keepdims=True)
    s1 = jnp.sum(x * x, 0, keepdims=True)
    m = s0 * (1.0 / G)
    v = s1 * (1.0 / G) - m * m
    al = lax.rsqrt(v + 1e-5)
    return al, -m * al + 1e-4


def _logsm(lo):
    lanes = lax.broadcasted_iota(i32, (G, H), 1)
    lom = jnp.where(lanes < C, lo, -1e30)
    mx = jnp.max(lom, axis=1, keepdims=True)
    ls = jnp.log(jnp.sum(jnp.exp(lom - mx), axis=1, keepdims=True))
    return lo - mx - ls


def _tc_readout_body(xcg_ref, xog_ref, W1c_ref, b1c_ref, W2c_ref, b2c_ref,
                     W1o_ref, b1o_ref, W2o_ref, b2o_ref,
                     W1t_ref, W1b_ref, b1co_ref, W2co_ref, b2co_ref,
                     oc_ref, oo_ref, oco_ref):
    xcg = xcg_ref[...]
    xog = xog_ref[...]

    def head(xg, W1r, b1r, W2r, b2r):
        al, bt = _gstats(xg)
        hh = jnp.maximum(jnp.dot(xg * al + bt, W1r,
                                 preferred_element_type=f32) + b1r, 0.0)
        al2, bt2 = _gstats(hh)
        return _logsm(jnp.dot(hh * al2 + bt2, W2r,
                              preferred_element_type=f32) + b2r)

    oc_ref[...] = head(xcg, W1c_ref[...], b1c_ref[...], W2c_ref[...],
                       b2c_ref[...])
    oo_ref[...] = head(xog, W1o_ref[...], b1o_ref[...], W2o_ref[...],
                       b2o_ref[...])
    alc, btc = _gstats(xcg)
    alo, bto = _gstats(xog)
    hh = jnp.maximum(
        jnp.dot(xcg * alc + btc, W1t_ref[...], preferred_element_type=f32)
        + jnp.dot(xog * alo + bto, W1b_ref[...], preferred_element_type=f32)
        + b1co_ref[...], 0.0)
    al2, bt2 = _gstats(hh)
    oco_ref[...] = _logsm(jnp.dot(hh * al2 + bt2, W2co_ref[...],
                                  preferred_element_type=f32) + b2co_ref[...])


_B_NH = pl.BlockSpec((RB, H), lambda i: (i, 0))
_B_2NH = pl.BlockSpec((2, RB, H), lambda i: (0, i, 0))
_B_S = pl.BlockSpec((8, H), lambda i: (0, 0))
_B_W = pl.BlockSpec((H, H), lambda i: (0, 0))
_B_B = pl.BlockSpec((1, H), lambda i: (0, 0))
_B_U = pl.BlockSpec((RB // 128, H), lambda i: (i, 0))
_B_G = pl.BlockSpec((G, H), lambda i: (0, 0))
_SNH = jax.ShapeDtypeStruct((NP, H), f32)
_S2NH = jax.ShapeDtypeStruct((2, NP, H), f32)
_SS = jax.ShapeDtypeStruct((8, H), f32)
_SU = jax.ShapeDtypeStruct((NP // H, H), f32)
_SG = jax.ShapeDtypeStruct((G, H), f32)


def _tc_stats(x):
    return pl.pallas_call(_tc_stats_body, grid=(NBLK,), in_specs=[_B_NH],
                          out_specs=_B_S, out_shape=_SS)(x)


def _tc_feat(x, sx, Wf, bf):
    return pl.pallas_call(
        _tc_feat_body, grid=(NBLK,),
        in_specs=[_B_NH, _B_S, _B_W, _B_B],
        out_specs=[_B_NH, _B_S], out_shape=[_SNH, _SS])(x, sx, Wf, bf)


def _tc_front(h, sh, degb, W):
    return pl.pallas_call(
        _tc_front_body, grid=(NBLK,),
        in_specs=[_B_NH, _B_S, _B_2NH, _B_W],
        out_specs=[_B_NH, _B_NH], out_shape=[_SNH, _SNH])(h, sh, degb, W)


def _tc_back(z, xw, degb, b):
    return pl.pallas_call(
        _tc_back_body, grid=(NBLK,),
        in_specs=[_B_2NH, _B_NH, _B_2NH, _B_B],
        out_specs=[_B_NH, _B_S], out_shape=[_SNH, _SS])(z, xw, degb, b)


def _tc_att(h, We_p, be_p, Wn_p, bnb_p):
    return pl.pallas_call(
        _tc_att_body, grid=(NBLK,),
        in_specs=[_B_NH, pl.BlockSpec((2 * H, H), lambda i: (0, 0)), _B_B,
                  _B_W, _B_B],
        out_specs=[_B_U, _B_U, _B_NH, _B_NH, _B_S, _B_S],
        out_shape=[_SU, _SU, _SNH, _SNH, _SS, _SS])(h, We_p, be_p, Wn_p, bnb_p)


def _tc_wfront(xc, xo, sc, so, dgw, Wc, Wo):
    return pl.pallas_call(
        _tc_wfront_body, grid=(NBLK,),
        in_specs=[_B_NH, _B_NH, _B_S, _B_S, _B_2NH, _B_W, _B_W],
        out_specs=[_B_2NH, _B_2NH],
        out_shape=[_S2NH, _S2NH])(xc, xo, sc, so, dgw, Wc, Wo)


def _tc_final(zw, xw, dgw, bc, bo, oh):
    return pl.pallas_call(
        _tc_final_body, grid=(NBLK,),
        in_specs=[_B_2NH, _B_2NH, _B_2NH, _B_B, _B_B,
                  pl.BlockSpec((RB, G), lambda i: (i, 0))],
        out_specs=[_B_G, _B_G], out_shape=[_SG, _SG])(zw, xw, dgw, bc, bo, oh)


def _tc_readout(xcg, xog, ws):
    gspec = pl.BlockSpec((G, H), lambda: (0, 0))
    bspec = pl.BlockSpec((1, H), lambda: (0, 0))
    specs = [gspec, gspec] + [gspec if w.shape[0] == H else bspec for w in ws]
    return pl.pallas_call(
        _tc_readout_body, grid=(),
        in_specs=specs, out_specs=[gspec, gspec, gspec],
        out_shape=[_SG, _SG, _SG])(xcg, xog, *ws)


def _sc_hist_body(row_hbm, ones_hbm, zeros_hbm, deg_out,
                  row_v, ones_v, deg_sh):
    cid = lax.axis_index("c")
    sid = lax.axis_index("s")
    wid = cid * 16 + sid
    r0 = sid * RPT
    pltpu.sync_copy(row_hbm.at[wid], row_v)
    pltpu.sync_copy(ones_hbm, ones_v)
    pltpu.sync_copy(zeros_hbm.at[pl.ds(r0, RPT)], deg_sh.at[pl.ds(r0, RPT)])
    plsc.subcore_barrier()

    def step(jb, c):
        pltpu.sync_copy(ones_v, deg_sh.at[row_v.at[jb]], add=True)
        return c

    lax.fori_loop(0, NCH, step, 0)
    plsc.subcore_barrier()
    pltpu.sync_copy(deg_sh.at[pl.ds(r0, RPT)], deg_out.at[cid, pl.ds(r0, RPT)])


def _sc_spmm_body(y_hbm, rc_hbm, zeros_hbm, z_out, IS, D, z_sh,
                  i0, i1, g0, g1, g2, g3, s0, s1, s2, s3):
    cid = lax.axis_index("c")
    sid = lax.axis_index("s")
    r0 = sid * RPZ
    isems = (i0, i1)
    gsems = (g0, g1, g2, g3)
    ssems = (s0, s1, s2, s3)

    def pipeline(nch, start):
        nsup = nch // SUP

        def idx_dma(s):
            pltpu.async_copy(rc_hbm.at[pl.ds(start + s * SUP, SUP)],
                             IS.at[s % 2], isems[s % 2])

        def wait_i(s):
            pltpu.make_async_copy(rc_hbm.at[pl.ds(0, SUP)], IS.at[s % 2],
                                  isems[s % 2]).wait()

        def gather(c, slot):
            s, j = divmod(c, SUP)
            pltpu.async_copy(y_hbm.at[IS.at[s % 2, j, 0]], D.at[slot],
                             gsems[slot])

        def wait_g(slot):
            pltpu.make_async_copy(y_hbm.at[IS.at[0, 0, 0]], D.at[slot],
                                  gsems[slot]).wait()

        def scatter(c):
            s, j = divmod(c, SUP)
            pltpu.async_copy(D.at[c % ND], z_sh.at[IS.at[s % 2, j, 1]],
                             ssems[c % ND], add=True)

        def wait_s(slot):
            pltpu.make_async_copy(D.at[slot], z_sh.at[IS.at[0, 0, 1]],
                                  ssems[slot]).wait()

        idx_dma(0)
        wait_i(0)
        gather(0, 0)
        gather(1, 1)
        swaited = set()

        def scatter_done(cc):
            if cc >= 0 and cc not in swaited:
                wait_s(cc % ND)
                swaited.add(cc)

        for c in range(nch):
            s, j = divmod(c, SUP)
            if j == 0 and s + 1 < nsup:
                scatter_done(c - 2)
                scatter_done(c - 1)
                idx_dma(s + 1)
            if j == SUP - 2 and s + 1 < nsup:
                wait_i(s + 1)
            scatter_done(c + 2 - ND)
            gather(min(c + 2, nch - 1), (c + 2) % ND)
            wait_g(c % ND)
            scatter(c)
        wait_g(nch % ND)
        wait_g((nch + 1) % ND)
        scatter_done(nch - 2)
        scatter_done(nch - 1)

    pltpu.sync_copy(zeros_hbm.at[pl.ds(r0, RPZ)], z_sh.at[pl.ds(r0, RPZ)])
    plsc.subcore_barrier()

    @pl.when(cid == 0)
    def _():
        pipeline(NCH0, sid * NCH0)

    @pl.when(cid == 1)
    def _():
        pipeline(NCH1, 16 * NCH0 + sid * NCH1)

    plsc.subcore_barrier()
    pltpu.sync_copy(z_sh.at[pl.ds(r0, RPZ)], z_out.at[cid, pl.ds(r0, RPZ)])

    @pl.when(sid == 15)
    def _():
        pltpu.sync_copy(zeros_hbm.at[pl.ds(0, NP - NPZ)],
                        z_out.at[cid, pl.ds(NPZ, NP - NPZ)])


def _sc_edge_body(u_hbm, v_hbm, rc_hbm, zeros_hbm, ec_out, degw_out,
                  u_v, v_v, IS, ecb, RW, deg_sh):
    cid = lax.axis_index("c")
    sid = lax.axis_index("s")
    wid = cid * 16 + sid
    r0 = sid * RPZ
    pltpu.sync_copy(u_hbm, u_v)
    pltpu.sync_copy(v_hbm, v_v)
    pltpu.sync_copy(zeros_hbm.at[pl.ds(r0, RPZ)], deg_sh.at[pl.ds(r0, RPZ)])
    plsc.subcore_barrier()

    def chunk(c, carry):
        s = c // SUP
        j = c - s * SUP

        @pl.when(j == 0)
        def _():
            pltpu.sync_copy(rc_hbm.at[wid, pl.ds(s * SUP, SUP)], IS)

        for g in range(K // 16):
            r16 = IS[j, 0, pl.ds(g * 16, 16)]
            c16 = IS[j, 1, pl.ds(g * 16, 16)]
            uu = plsc.load_gather(u_v, [r16])
            vv = plsc.load_gather(v_v, [c16])
            ec = 1.0 / (1.0 + jnp.exp(-(uu + vv)))
            ecb[pl.ds(g * 16, 16)] = ec

        def edge(e, cc):
            w16 = plsc.load_gather(ecb, [jnp.full((16,), e, i32)])
            w16o = 1.0 - w16
            for q in range(4):
                RW[e, pl.ds(q * 16, 16)] = w16
            for q in range(4, 8):
                RW[e, pl.ds(q * 16, 16)] = w16o
            del w16, w16o
            return cc

        lax.fori_loop(0, K, edge, 0)
        pltpu.sync_copy(ecb, ec_out.at[wid, c])
        pltpu.sync_copy(RW, deg_sh.at[IS.at[j, 0]], add=True)
        return carry

    lax.fori_loop(0, NCH, chunk, 0)
    plsc.subcore_barrier()
    pltpu.sync_copy(deg_sh.at[pl.ds(r0, RPZ)], degw_out.at[cid, pl.ds(r0, RPZ)])

    @pl.when(sid == 15)
    def _():
        pltpu.sync_copy(zeros_hbm.at[pl.ds(0, NP - NPZ)],
                        degw_out.at[cid, pl.ds(NPZ, NP - NPZ)])


def _sc_wspmm_body(y_hbm, rcb_hbm, ew_hbm, zeros_hbm, z_out, IS, EW, D, z_sh,
                   i0, i1, e0, e1, g0, g1, g2, g3, s0, s1, s2, s3):
    cid = lax.axis_index("c")
    sid = lax.axis_index("s")
    r0 = sid * RPZ
    isems = (i0, i1)
    esems = (e0, e1)
    gsems = (g0, g1, g2, g3)
    ssems = (s0, s1, s2, s3)
    fv = jnp.full((16,), cid.astype(f32), f32)
    a0 = fv
    a1 = 1.0 - 2.0 * fv

    def idx_dma(sb, s):
        pltpu.async_copy(rcb_hbm.at[cid, sid, pl.ds(s * SUPW, SUPW)],
                         IS.at[sb], isems[sb])
        pltpu.async_copy(ew_hbm.at[sid, pl.ds(s * SUPW, SUPW)],
                         EW.at[sb], esems[sb])

    def wait_i(sb):
        pltpu.make_async_copy(rcb_hbm.at[0, 0, pl.ds(0, SUPW)], IS.at[sb],
                              isems[sb]).wait()
        pltpu.make_async_copy(ew_hbm.at[0, pl.ds(0, SUPW)], EW.at[sb],
                              esems[sb]).wait()

    def gather(slot, s2, j):
        pltpu.async_copy(y_hbm.at[IS.at[s2, j, 0]], D.at[slot], gsems[slot])

    def wait_g(slot):
        pltpu.make_async_copy(y_hbm.at[IS.at[0, 0, 0]], D.at[slot],
                              gsems[slot]).wait()

    def scatter(b, s2, j):
        pltpu.async_copy(D.at[b], z_sh.at[IS.at[s2, j, 1]], ssems[b],
                         add=True)

    def wait_s(slot):
        pltpu.make_async_copy(D.at[slot], z_sh.at[IS.at[0, 0, 1]],
                              ssems[slot]).wait()

    idx_dma(0, 0)
    pltpu.sync_copy(zeros_hbm.at[pl.ds(r0, RPZ)], z_sh.at[pl.ds(r0, RPZ)])
    plsc.subcore_barrier()
    wait_i(0)
    gather(0, 0, 0)
    gather(1, 0, 1)

    def triple(t, carry):
        c0 = 3 * t
        for b in range(3):
            c = c0 + b
            s = c // SUPW
            j = c - s * SUPW
            s2 = s % 2
            if b >= 1:
                wait_s((b + 2) % 3)
            else:
                @pl.when(t > 0)
                def _():
                    wait_s(2)
            for kk in range(2):
                @pl.when((j == 0) & (s + 1 < NSUPW) & ((s + 1) % 2 == kk))
                def _():
                    idx_dma(kk, s + 1)
            for kk in range(2):
                @pl.when((j == SUPW - 2) & (s + 1 < NSUPW)
                         & ((s + 1) % 2 == kk))
                def _():
                    wait_i(kk)
            p = jnp.minimum(c + 2, NCHW - 1)
            sp = p // SUPW
            gather((b + 2) % 3, sp % 2, p - sp * SUPW)
            wait_g(b)
            sv = jnp.full((16,), s2, i32)
            jv = jnp.full((16,), j, i32)

            def rbody(r, cc):
                w16 = plsc.load_gather(EW, [sv, jv, jnp.full((16,), r, i32)])
                w16 = a0 + a1 * w16
                for fch in range(8):
                    sl = D[b, r, pl.ds(fch * 16, 16)]
                    D[b, r, pl.ds(fch * 16, 16)] = sl * w16
                return cc

            lax.fori_loop(0, K, rbody, 0)
            scatter(b, s2, j)
        return carry

    lax.fori_loop(0, NCHW // 3, triple, 0)
    wait_g(0)
    wait_g(1)
    wait_s(2)
    plsc.subcore_barrier()
    pltpu.sync_copy(z_sh.at[pl.ds(r0, RPZ)], z_out.at[cid, pl.ds(r0, RPZ)])

    @pl.when(sid == 15)
    def _():
        pltpu.sync_copy(zeros_hbm.at[pl.ds(0, NP - NPZ)],
                        z_out.at[cid, pl.ds(NPZ, NP - NPZ)])


def _mk_mesh():
    return plsc.VectorSubcoreMesh(core_axis_name="c", subcore_axis_name="s")


def _sc_hist(row_t, ones_kh, zeros_nph):
    k = functools.partial(
        pl.kernel,
        compiler_params=pltpu.CompilerParams(needs_layout_passes=False),
        out_type=jax.ShapeDtypeStruct((2, NP, H), f32),
        mesh=_mk_mesh(),
        scratch_types=[pltpu.VMEM((NCH, K), i32), pltpu.VMEM((K, H), f32),
                       pltpu.VMEM_SHARED((NP, H), f32)])(_sc_hist_body)
    return k(row_t, ones_kh, zeros_nph)


def _sc_spmm(y, rc_t, zeros_nph):
    k = functools.partial(
        pl.kernel,
        compiler_params=pltpu.CompilerParams(needs_layout_passes=False),
        out_type=jax.ShapeDtypeStruct((2, NP, H), f32),
        mesh=_mk_mesh(),
        scratch_types=[pltpu.VMEM((2, SUP, 2, K), i32),
                       pltpu.VMEM((ND, K, H), f32),
                       pltpu.VMEM_SHARED((NPZ, H), f32)]
        + [pltpu.SemaphoreType.DMA] * 10)(_sc_spmm_body)
    return k(y, rc_t, zeros_nph)


def _sc_edge(u, v, rc_t, zeros_nph):
    k = functools.partial(
        pl.kernel,
        compiler_params=pltpu.CompilerParams(needs_layout_passes=False),
        out_type=(jax.ShapeDtypeStruct((NT, NCH, K), f32),
                  jax.ShapeDtypeStruct((2, NP, H), f32)),
        mesh=_mk_mesh(),
        scratch_types=[pltpu.VMEM((NP,), f32), pltpu.VMEM((NP,), f32),
                       pltpu.VMEM((SUP, 2, K), i32), pltpu.VMEM((K,), f32),
                       pltpu.VMEM((K, H), f32),
                       pltpu.VMEM_SHARED((NPZ, H), f32)])(
        _sc_edge_body)
    return k(u, v, rc_t, zeros_nph)


def _sc_wspmm(y2, rcb, ew, zeros_nph):
    k = functools.partial(
        pl.kernel,
        compiler_params=pltpu.CompilerParams(needs_layout_passes=False),
        out_type=jax.ShapeDtypeStruct((2, NP, H), f32),
        mesh=_mk_mesh(),
        scratch_types=[pltpu.VMEM((2, SUPW, 2, K), i32),
                       pltpu.VMEM((2, SUPW, K), f32),
                       pltpu.VMEM((3, K, H), f32),
                       pltpu.VMEM_SHARED((NPZ, H), f32)]
        + [pltpu.SemaphoreType.DMA] * 12)(_sc_wspmm_body)
    return k(y2, rcb, ew, zeros_nph)


def kernel(x, W_feat, b_feat, W0, b0, W1, b1, W2, b2, We, be, Wn, bn_b,
           Wc, bc, Wo, bo, W1c, b1c, W2c, b2c, W1o, b1o, W2o, b2o,
           W1co, b1co, W2co, b2co, edge_index, batch):
    row = edge_index[0].astype(i32)
    col = edge_index[1].astype(i32)
    padn = jnp.full((NT * NCH * K - E,), N, i32)
    row_t = jnp.concatenate([row, padn]).reshape(NT, NCH, K)
    col_t = jnp.concatenate([col, padn]).reshape(NT, NCH, K)
    rc_t = jnp.stack([row_t, col_t], axis=2)
    xp = jnp.zeros((NP, x.shape[1]), f32).at[:N].set(x)
    zeros_nph = jnp.zeros((NP, H), f32)
    ones_kh = jnp.ones((K, H), f32)
    rowv = lambda a: a.reshape(1, H)
    bf = rowv(b_feat)
    We_p = jnp.zeros((2 * H, H), f32).at[:, :2].set(We)
    be_p = jnp.zeros((1, H), f32).at[0, :2].set(be)
    Wn_p = jnp.zeros((H, H), f32).at[:, :2].set(Wn)
    bnb_p = jnp.zeros((1, H), f32).at[0, :2].set(bn_b)
    bp = jnp.concatenate([batch.astype(i32), jnp.full((NP - N,), G, i32)])
    oh = (bp[:, None] == jnp.arange(G, dtype=i32)[None, :]).astype(f32)
    W2c_p = jnp.zeros((H, H), f32).at[:, :C].set(W2c)
    b2c_p = jnp.zeros((1, H), f32).at[0, :C].set(b2c)
    W2o_p = jnp.zeros((H, H), f32).at[:, :C].set(W2o)
    b2o_p = jnp.zeros((1, H), f32).at[0, :C].set(b2o)
    W2co_p = jnp.zeros((H, H), f32).at[:, :C].set(W2co)
    b2co_p = jnp.zeros((1, H), f32).at[0, :C].set(b2co)

    degu = _sc_hist(row_t, ones_kh, zeros_nph)
    sx = _tc_stats(xp)
    h, sh = _tc_feat(xp, sx, W_feat, bf)

    rc_sp = rc_t.reshape(NT * NCH, 2, K)
    for (Wi, bi) in ((W0, b0), (W1, b1), (W2, b2)):
        xw, y = _tc_front(h, sh, degu, Wi)
        z = _sc_spmm(y, rc_sp, zeros_nph)
        h, sh = _tc_back(z, xw, degu, rowv(bi))

    u, v, xc, xo, sxc, sxo = _tc_att(h, We_p, be_p, Wn_p, bnb_p)
    ec, degw = _sc_edge(u.reshape(NP), v.reshape(NP), rc_t, zeros_nph)

    xww, yw = _tc_wfront(xc, xo, sxc, sxo, degw, Wc, Wo)
    y2 = yw.reshape(2 * NP, H)
    NCHP = NSUPW * SUPW
    cpad = jnp.full((16, NCHP - NCHW, K), N, i32)
    row_w = jnp.concatenate([row_t.reshape(16, NCHW, K), cpad], axis=1)
    col_w = jnp.concatenate([col_t.reshape(16, NCHW, K), cpad], axis=1)
    rcb = jnp.stack([jnp.stack([row_w, col_w], axis=2),
                     jnp.stack([row_w + NP, col_w], axis=2)], axis=0)
    ew_w = jnp.concatenate(
        [ec.reshape(16, NCHW, K), jnp.zeros((16, NCHP - NCHW, K), f32)],
        axis=1)
    zw = _sc_wspmm(y2, rcb, ew_w, zeros_nph)

    xcg, xog = _tc_final(zw, xww, degw, rowv(bc), rowv(bo), oh)
    ws = (W1c, rowv(b1c), W2c_p, b2c_p, W1o, rowv(b1o), W2o_p, b2o_p,
          W1co[:H], W1co[H:], rowv(b1co), W2co_p, b2co_p)
    oc, oo, oco = _tc_readout(xcg, xog, ws)
    return (oc[:, :C], oo[:, :C], oco[:, :C])


# speedup vs baseline: 1.0948x; 1.0085x over previous
import functools

import jax
import jax.numpy as jnp
from jax import lax
from jax.experimental import pallas as pl
from jax.experimental.pallas import tpu as pltpu
from jax.experimental.pallas import tpu_sc as plsc

N = 10000
E = 320000
H = 128
C = 10
G = 128

NP = 10240
NPZ = 10112
K = 112
NCH = 90
NT = 32
NCHW = 2 * NCH
SUP = 6
SUPW = 8
NSUPW = 23
ND = 3
NCH0 = 126
NCH1 = 2 * NCH - NCH0
RB = 1024
NBLK = NP // RB
RPT = NP // 16
RPZ = NPZ // 16

f32 = jnp.float32
i32 = jnp.int32


def _rowmask(i):
    rows = lax.broadcasted_iota(i32, (RB, 1), 0) + i * RB
    return (rows < N).astype(f32)


def _affine(s_ref):
    s0 = s_ref[0:1, :]
    s1 = s_ref[1:2, :]
    m = s0 * (1.0 / N)
    v = s1 * (1.0 / N) - m * m
    al = lax.rsqrt(v + 1e-5)
    return al, -m * al + 1e-4


def _stats_of(x, i, s_ref):
    xm = x * _rowmask(i)
    part = jnp.concatenate(
        [jnp.sum(xm, 0, keepdims=True), jnp.sum(xm * xm, 0, keepdims=True),
         jnp.zeros((6, H), f32)], axis=0)

    @pl.when(i == 0)
    def _():
        s_ref[...] = part

    @pl.when(i > 0)
    def _():
        s_ref[...] = s_ref[...] + part


def _tc_stats_body(x_ref, s_ref):
    _stats_of(x_ref[...], pl.program_id(0), s_ref)


def _tc_feat_body(x_ref, sx_ref, Wf_ref, bf_ref, h_ref, s1_ref):
    i = pl.program_id(0)
    al, bt = _affine(sx_ref)
    xn = x_ref[...] * al + bt
    h = jnp.maximum(jnp.dot(xn, Wf_ref[...], preferred_element_type=f32)
                    + bf_ref[...], 0.0)
    h_ref[...] = h
    _stats_of(h, i, s1_ref)


def _tc_front_body(h_ref, sh_ref, deg_ref, W_ref, xw_ref, y_ref):
    i = pl.program_id(0)
    al, bt = _affine(sh_ref)
    xn = h_ref[...] * al + bt
    xw = jnp.dot(xn, W_ref[...], preferred_element_type=f32)
    dis = lax.rsqrt(deg_ref[0] + deg_ref[1] + 1.0)
    xw_ref[...] = xw
    y_ref[...] = dis * xw * _rowmask(i)


def _tc_back_body(z_ref, xw_ref, deg_ref, b_ref, h_ref, s_ref):
    i = pl.program_id(0)
    dis = lax.rsqrt(deg_ref[0] + deg_ref[1] + 1.0)
    xw = xw_ref[...]
    h = jnp.maximum(dis * (z_ref[0] + z_ref[1]) + dis * dis * xw + b_ref[...],
                    0.0)
    h_ref[...] = h
    _stats_of(h, i, s_ref)


def _tc_att_body(h_ref, We_ref, be_ref, Wn_ref, bnb_ref,
                 u_ref, v_ref, xc_ref, xo_ref, sc_ref, so_ref):
    i = pl.program_id(0)
    hb = h_ref[...]
    lanes = lax.broadcasted_iota(i32, (1, H), 1)
    sel = jnp.where(lanes == 0, 1.0, jnp.where(lanes == 1, -1.0, 0.0))
    wuv = lax.dot_general(sel, We_ref[...], (((1,), (1,)), ((), ())),
                          preferred_element_type=f32)
    wu = wuv[:, 0:H]
    wv = wuv[:, H:2 * H]
    be_d = jnp.sum(be_ref[...] * sel, axis=1, keepdims=True)
    urows = []
    vrows = []
    for sb in range(RB // 128):
        hs = hb[sb * 128:(sb + 1) * 128, :]
        urows.append(lax.dot_general(wu, hs, (((1,), (1,)), ((), ())),
                                     preferred_element_type=f32) + be_d)
        vrows.append(lax.dot_general(wv, hs, (((1,), (1,)), ((), ())),
                                     preferred_element_type=f32))
    u_ref[...] = jnp.concatenate(urows, axis=0)
    v_ref[...] = jnp.concatenate(vrows, axis=0)
    nl = jnp.dot(hb, Wn_ref[...], preferred_element_type=f32) + bnb_ref[...]
    d01 = nl[:, 0:1] - nl[:, 1:2]
    att0 = 1.0 / (1.0 + jnp.exp(-d01))
    xc = att0 * hb
    xo = hb - xc
    xc_ref[...] = xc
    xo_ref[...] = xo
    _stats_of(xc, i, sc_ref)
    _stats_of(xo, i, so_ref)


def _split_dis(dgw_ref):
    dsum = dgw_ref[0] + dgw_ref[1]
    ri = lax.broadcasted_iota(i32, (H, H), 0)
    s0 = (ri == 0).astype(f32)
    s64 = (ri == 64).astype(f32)
    degc = jnp.dot(dsum, s0, preferred_element_type=f32)
    dego = jnp.dot(dsum, s64, preferred_element_type=f32)
    return lax.rsqrt(degc + 1.0), lax.rsqrt(dego + 1.0)


def _tc_wfront_body(xc_ref, xo_ref, sc_ref, so_ref, dgw_ref,
                    Wc_ref, Wo_ref, xw_ref, y_ref):
    i = pl.program_id(0)
    msk = _rowmask(i)
    disc, diso = _split_dis(dgw_ref)
    alc, btc = _affine(sc_ref)
    xwc = jnp.dot(xc_ref[...] * alc + btc, Wc_ref[...],
                  preferred_element_type=f32)
    alo, bto = _affine(so_ref)
    xwo = jnp.dot(xo_ref[...] * alo + bto, Wo_ref[...],
                  preferred_element_type=f32)
    xw_ref[0] = xwc
    xw_ref[1] = xwo
    y_ref[0] = disc * xwc * msk
    y_ref[1] = diso * xwo * msk


def _tc_final_body(zw_ref, xw_ref, dgw_ref, bc_ref, bo_ref, oh_ref,
                   pc_ref, po_ref):
    i = pl.program_id(0)
    disc, diso = _split_dis(dgw_ref)
    xc2 = jnp.maximum(disc * zw_ref[0] + disc * disc * xw_ref[0] + bc_ref[...],
                      0.0)
    xo2 = jnp.maximum(diso * zw_ref[1] + diso * diso * xw_ref[1] + bo_ref[...],
                      0.0)
    oh = oh_ref[...]
    pc = lax.dot_general(oh, xc2, (((0,), (0,)), ((), ())),
                         preferred_element_type=f32)
    po = lax.dot_general(oh, xo2, (((0,), (0,)), ((), ())),
                         preferred_element_type=f32)

    @pl.when(i == 0)
    def _():
        pc_ref[...] = pc
        po_ref[...] = po

    @pl.when(i > 0)
    def _():
        pc_ref[...] = pc_ref[...] + pc
        po_ref[...] = po_ref[...] + po


def _gstats(x):
    s0 = jnp.sum(x, 0, keepdims=True)
    s1 = jnp.sum(x * x, 0, keepdims=True)
    m = s0 * (1.0 / G)
    v = s1 * (1.0 / G) - m * m
    al = lax.rsqrt(v + 1e-5)
    return al, -m * al + 1e-4


def _logsm(lo):
    lanes = lax.broadcasted_iota(i32, (G, H), 1)
    lom = jnp.where(lanes < C, lo, -1e30)
    mx = jnp.max(lom, axis=1, keepdims=True)
    ls = jnp.log(jnp.sum(jnp.exp(lom - mx), axis=1, keepdims=True))
    return lo - mx - ls


def _tc_readout_body(xcg_ref, xog_ref, W1c_ref, b1c_ref, W2c_ref, b2c_ref,
                     W1o_ref, b1o_ref, W2o_ref, b2o_ref,
                     W1t_ref, W1b_ref, b1co_ref, W2co_ref, b2co_ref,
                     oc_ref, oo_ref, oco_ref):
    xcg = xcg_ref[...]
    xog = xog_ref[...]

    def head(xg, W1r, b1r, W2r, b2r):
        al, bt = _gstats(xg)
        hh = jnp.maximum(jnp.dot(xg * al + bt, W1r,
                                 preferred_element_type=f32) + b1r, 0.0)
        al2, bt2 = _gstats(hh)
        return _logsm(jnp.dot(hh * al2 + bt2, W2r,
                              preferred_element_type=f32) + b2r)

    oc_ref[...] = head(xcg, W1c_ref[...], b1c_ref[...], W2c_ref[...],
                       b2c_ref[...])
    oo_ref[...] = head(xog, W1o_ref[...], b1o_ref[...], W2o_ref[...],
                       b2o_ref[...])
    alc, btc = _gstats(xcg)
    alo, bto = _gstats(xog)
    hh = jnp.maximum(
        jnp.dot(xcg * alc + btc, W1t_ref[...], preferred_element_type=f32)
        + jnp.dot(xog * alo + bto, W1b_ref[...], preferred_element_type=f32)
        + b1co_ref[...], 0.0)
    al2, bt2 = _gstats(hh)
    oco_ref[...] = _logsm(jnp.dot(hh * al2 + bt2, W2co_ref[...],
                                  preferred_element_type=f32) + b2co_ref[...])


_B_NH = pl.BlockSpec((RB, H), lambda i: (i, 0))
_B_2NH = pl.BlockSpec((2, RB, H), lambda i: (0, i, 0))
_B_S = pl.BlockSpec((8, H), lambda i: (0, 0))
_B_W = pl.BlockSpec((H, H), lambda i: (0, 0))
_B_B = pl.BlockSpec((1, H), lambda i: (0, 0))
_B_U = pl.BlockSpec((RB // 128, H), lambda i: (i, 0))
_B_G = pl.BlockSpec((G, H), lambda i: (0, 0))
_SNH = jax.ShapeDtypeStruct((NP, H), f32)
_S2NH = jax.ShapeDtypeStruct((2, NP, H), f32)
_SS = jax.ShapeDtypeStruct((8, H), f32)
_SU = jax.ShapeDtypeStruct((NP // H, H), f32)
_SG = jax.ShapeDtypeStruct((G, H), f32)


def _tc_stats(x):
    return pl.pallas_call(_tc_stats_body, grid=(NBLK,), in_specs=[_B_NH],
                          out_specs=_B_S, out_shape=_SS)(x)


def _tc_feat(x, sx, Wf, bf):
    return pl.pallas_call(
        _tc_feat_body, grid=(NBLK,),
        in_specs=[_B_NH, _B_S, _B_W, _B_B],
        out_specs=[_B_NH, _B_S], out_shape=[_SNH, _SS])(x, sx, Wf, bf)


def _tc_front(h, sh, degb, W):
    return pl.pallas_call(
        _tc_front_body, grid=(NBLK,),
        in_specs=[_B_NH, _B_S, _B_2NH, _B_W],
        out_specs=[_B_NH, _B_NH], out_shape=[_SNH, _SNH])(h, sh, degb, W)


def _tc_back(z, xw, degb, b):
    return pl.pallas_call(
        _tc_back_body, grid=(NBLK,),
        in_specs=[_B_2NH, _B_NH, _B_2NH, _B_B],
        out_specs=[_B_NH, _B_S], out_shape=[_SNH, _SS])(z, xw, degb, b)


def _tc_att(h, We_p, be_p, Wn_p, bnb_p):
    return pl.pallas_call(
        _tc_att_body, grid=(NBLK,),
        in_specs=[_B_NH, pl.BlockSpec((2 * H, H), lambda i: (0, 0)), _B_B,
                  _B_W, _B_B],
        out_specs=[_B_U, _B_U, _B_NH, _B_NH, _B_S, _B_S],
        out_shape=[_SU, _SU, _SNH, _SNH, _SS, _SS])(h, We_p, be_p, Wn_p, bnb_p)


def _tc_wfront(xc, xo, sc, so, dgw, Wc, Wo):
    return pl.pallas_call(
        _tc_wfront_body, grid=(NBLK,),
        in_specs=[_B_NH, _B_NH, _B_S, _B_S, _B_2NH, _B_W, _B_W],
        out_specs=[_B_2NH, _B_2NH],
        out_shape=[_S2NH, _S2NH])(xc, xo, sc, so, dgw, Wc, Wo)


def _tc_final(zw, xw, dgw, bc, bo, oh):
    return pl.pallas_call(
        _tc_final_body, grid=(NBLK,),
        in_specs=[_B_2NH, _B_2NH, _B_2NH, _B_B, _B_B,
                  pl.BlockSpec((RB, G), lambda i: (i, 0))],
        out_specs=[_B_G, _B_G], out_shape=[_SG, _SG])(zw, xw, dgw, bc, bo, oh)


def _tc_readout(xcg, xog, ws):
    gspec = pl.BlockSpec((G, H), lambda: (0, 0))
    bspec = pl.BlockSpec((1, H), lambda: (0, 0))
    specs = [gspec, gspec] + [gspec if w.shape[0] == H else bspec for w in ws]
    return pl.pallas_call(
        _tc_readout_body, grid=(),
        in_specs=specs, out_specs=[gspec, gspec, gspec],
        out_shape=[_SG, _SG, _SG])(xcg, xog, *ws)


def _sc_hist_body(row_hbm, ones_hbm, zeros_hbm, deg_out,
                  row_v, ones_v, deg_sh):
    cid = lax.axis_index("c")
    sid = lax.axis_index("s")
    wid = cid * 16 + sid
    r0 = sid * RPT
    pltpu.sync_copy(row_hbm.at[wid], row_v)
    pltpu.sync_copy(ones_hbm, ones_v)
    pltpu.sync_copy(zeros_hbm.at[pl.ds(r0, RPT)], deg_sh.at[pl.ds(r0, RPT)])
    plsc.subcore_barrier()

    def step(jb, c):
        pltpu.sync_copy(ones_v, deg_sh.at[row_v.at[jb]], add=True)
        return c

    lax.fori_loop(0, NCH, step, 0)
    plsc.subcore_barrier()
    pltpu.sync_copy(deg_sh.at[pl.ds(r0, RPT)], deg_out.at[cid, pl.ds(r0, RPT)])


def _sc_spmm_body(y_hbm, rc_hbm, zeros_hbm, z_out, IS, D, z_sh,
                  i0, i1, g0, g1, g2, g3, s0, s1, s2, s3):
    cid = lax.axis_index("c")
    sid = lax.axis_index("s")
    r0 = sid * RPZ
    isems = (i0, i1)
    gsems = (g0, g1, g2, g3)
    ssems = (s0, s1, s2, s3)

    def pipeline(nch, start):
        nsup = nch // SUP

        def idx_dma(s):
            pltpu.async_copy(rc_hbm.at[pl.ds(start + s * SUP, SUP)],
                             IS.at[s % 2], isems[s % 2])

        def wait_i(s):
            pltpu.make_async_copy(rc_hbm.at[pl.ds(0, SUP)], IS.at[s % 2],
                                  isems[s % 2]).wait()

        def gather(c, slot):
            s, j = divmod(c, SUP)
            pltpu.async_copy(y_hbm.at[IS.at[s % 2, j, 0]], D.at[slot],
                             gsems[slot])

        def wait_g(slot):
            pltpu.make_async_copy(y_hbm.at[IS.at[0, 0, 0]], D.at[slot],
                                  gsems[slot]).wait()

        def scatter(c):
            s, j = divmod(c, SUP)
            pltpu.async_copy(D.at[c % ND], z_sh.at[IS.at[s % 2, j, 1]],
                             ssems[c % ND], add=True)

        def wait_s(slot):
            pltpu.make_async_copy(D.at[slot], z_sh.at[IS.at[0, 0, 1]],
                                  ssems[slot]).wait()

        idx_dma(0)
        wait_i(0)
        gather(0, 0)
        gather(1, 1)
        swaited = set()

        def scatter_done(cc):
            if cc >= 0 and cc not in swaited:
                wait_s(cc % ND)
                swaited.add(cc)

        for c in range(nch):
            s, j = divmod(c, SUP)
            if j == 0 and s + 1 < nsup:
                scatter_done(c - 2)
                scatter_done(c - 1)
                idx_dma(s + 1)
            if j == SUP - 2 and s + 1 < nsup:
                wait_i(s + 1)
            scatter_done(c + 2 - ND)
            gather(min(c + 2, nch - 1), (c + 2) % ND)
            wait_g(c % ND)
            scatter(c)
        wait_g(nch % ND)
        wait_g((nch + 1) % ND)
        scatter_done(nch - 2)
        scatter_done(nch - 1)

    pltpu.sync_copy(zeros_hbm.at[pl.ds(r0, RPZ)], z_sh.at[pl.ds(r0, RPZ)])
    plsc.subcore_barrier()

    @pl.when(cid == 0)
    def _():
        pipeline(NCH0, sid * NCH0)

    @pl.when(cid == 1)
    def _():
        pipeline(NCH1, 16 * NCH0 + sid * NCH1)

    plsc.subcore_barrier()
    pltpu.sync_copy(z_sh.at[pl.ds(r0, RPZ)], z_out.at[cid, pl.ds(r0, RPZ)])

    @pl.when(sid == 15)
    def _():
        pltpu.sync_copy(zeros_hbm.at[pl.ds(0, NP - NPZ)],
                        z_out.at[cid, pl.ds(NPZ, NP - NPZ)])


def _sc_edge_body(u_hbm, v_hbm, rc_hbm, zeros_hbm, ec_out, degw_out,
                  u_v, v_v, IS, ecb, RW, deg_sh):
    cid = lax.axis_index("c")
    sid = lax.axis_index("s")
    wid = cid * 16 + sid
    r0 = sid * RPZ
    pltpu.sync_copy(u_hbm, u_v)
    pltpu.sync_copy(v_hbm, v_v)
    pltpu.sync_copy(zeros_hbm.at[pl.ds(r0, RPZ)], deg_sh.at[pl.ds(r0, RPZ)])
    plsc.subcore_barrier()

    def chunk(c, carry):
        s = c // SUP
        j = c - s * SUP

        @pl.when(j == 0)
        def _():
            pltpu.sync_copy(rc_hbm.at[wid, pl.ds(s * SUP, SUP)], IS)

        for g in range(K // 16):
            r16 = IS[j, 0, pl.ds(g * 16, 16)]
            c16 = IS[j, 1, pl.ds(g * 16, 16)]
            uu = plsc.load_gather(u_v, [r16])
            vv = plsc.load_gather(v_v, [c16])
            ec = 1.0 / (1.0 + jnp.exp(-(uu + vv)))
            ecb[pl.ds(g * 16, 16)] = ec

        def edge(e, cc):
            w16 = plsc.load_gather(ecb, [jnp.full((16,), e, i32)])
            w16o = 1.0 - w16
            for q in range(4):
                RW[e, pl.ds(q * 16, 16)] = w16
            for q in range(4, 8):
                RW[e, pl.ds(q * 16, 16)] = w16o
            del w16, w16o
            return cc

        lax.fori_loop(0, K, edge, 0)
        pltpu.sync_copy(ecb, ec_out.at[wid, c])
        pltpu.sync_copy(RW, deg_sh.at[IS.at[j, 0]], add=True)
        return carry

    lax.fori_loop(0, NCH, chunk, 0)
    plsc.subcore_barrier()
    pltpu.sync_copy(deg_sh.at[pl.ds(r0, RPZ)], degw_out.at[cid, pl.ds(r0, RPZ)])

    @pl.when(sid == 15)
    def _():
        pltpu.sync_copy(zeros_hbm.at[pl.ds(0, NP - NPZ)],
                        degw_out.at[cid, pl.ds(NPZ, NP - NPZ)])


def _sc_wspmm_body(y_hbm, rcb_hbm, ew_hbm, zeros_hbm, z_out, IS, EW, D, z_sh,
                   i0, i1, e0, e1, g0, g1, g2, g3, s0, s1, s2, s3):
    cid = lax.axis_index("c")
    sid = lax.axis_index("s")
    r0 = sid * RPZ
    isems = (i0, i1)
    esems = (e0, e1)
    gsems = (g0, g1, g2, g3)
    ssems = (s0, s1, s2, s3)
    fv = jnp.full((16,), cid.astype(f32), f32)
    a0 = fv
    a1 = 1.0 - 2.0 * fv

    def idx_dma(sb, s):
        pltpu.async_copy(rcb_hbm.at[cid, sid, pl.ds(s * SUPW, SUPW)],
                         IS.at[sb], isems[sb])
        pltpu.async_copy(ew_hbm.at[sid, pl.ds(s * SUPW, SUPW)],
                         EW.at[sb], esems[sb])

    def wait_i(sb):
        pltpu.make_async_copy(rcb_hbm.at[0, 0, pl.ds(0, SUPW)], IS.at[sb],
                              isems[sb]).wait()
        pltpu.make_async_copy(ew_hbm.at[0, pl.ds(0, SUPW)], EW.at[sb],
                              esems[sb]).wait()

    def gather(slot, s2, j):
        pltpu.async_copy(y_hbm.at[IS.at[s2, j, 0]], D.at[slot], gsems[slot])

    def wait_g(slot):
        pltpu.make_async_copy(y_hbm.at[IS.at[0, 0, 0]], D.at[slot],
                              gsems[slot]).wait()

    def scatter(b, s2, j):
        pltpu.async_copy(D.at[b], z_sh.at[IS.at[s2, j, 1]], ssems[b],
                         add=True)

    def wait_s(slot):
        pltpu.make_async_copy(D.at[slot], z_sh.at[IS.at[0, 0, 1]],
                              ssems[slot]).wait()

    idx_dma(0, 0)
    pltpu.sync_copy(zeros_hbm.at[pl.ds(r0, RPZ)], z_sh.at[pl.ds(r0, RPZ)])
    plsc.subcore_barrier()
    wait_i(0)
    gather(0, 0, 0)
    gather(1, 0, 1)

    def triple(t, carry):
        c0 = 3 * t
        for b in range(3):
            c = c0 + b
            s = c // SUPW
            j = c - s * SUPW
            s2 = s % 2
            if b >= 1:
                wait_s((b + 2) % 3)
            else:
                @pl.when(t > 0)
                def _():
                    wait_s(2)
            for kk in range(2):
                @pl.when((j == 0) & (s + 1 < NSUPW) & ((s + 1) % 2 == kk))
                def _():
                    idx_dma(kk, s + 1)
            for kk in range(2):
                @pl.when((j == SUPW - 2) & (s + 1 < NSUPW)
                         & ((s + 1) % 2 == kk))
                def _():
                    wait_i(kk)
            p = jnp.minimum(c + 2, NCHW - 1)
            sp = p // SUPW
            gather((b + 2) % 3, sp % 2, p - sp * SUPW)
            wait_g(b)
            sv = jnp.full((16,), s2, i32)
            jv = jnp.full((16,), j, i32)

            def rbody(r, cc):
                w16 = plsc.load_gather(EW, [sv, jv, jnp.full((16,), r, i32)])
                w16 = a0 + a1 * w16
                for fch in range(8):
                    sl = D[b, r, pl.ds(fch * 16, 16)]
                    D[b, r, pl.ds(fch * 16, 16)] = sl * w16
                return cc

            lax.fori_loop(0, K, rbody, 0)
            scatter(b, s2, j)
        return carry

    lax.fori_loop(0, NCHW // 3, triple, 0)
    wait_g(0)
    wait_g(1)
    wait_s(2)
    plsc.subcore_barrier()
    pltpu.sync_copy(z_sh.at[pl.ds(r0, RPZ)], z_out.at[cid, pl.ds(r0, RPZ)])

    @pl.when(sid == 15)
    def _():
        pltpu.sync_copy(zeros_hbm.at[pl.ds(0, NP - NPZ)],
                        z_out.at[cid, pl.ds(NPZ, NP - NPZ)])


def _mk_mesh():
    return plsc.VectorSubcoreMesh(core_axis_name="c", subcore_axis_name="s")


def _sc_hist(row_t, ones_kh, zeros_nph):
    k = functools.partial(
        pl.kernel,
        compiler_params=pltpu.CompilerParams(needs_layout_passes=False),
        out_type=jax.ShapeDtypeStruct((2, NP, H), f32),
        mesh=_mk_mesh(),
        scratch_types=[pltpu.VMEM((NCH, K), i32), pltpu.VMEM((K, H), f32),
                       pltpu.VMEM_SHARED((NP, H), f32)])(_sc_hist_body)
    return k(row_t, ones_kh, zeros_nph)


def _sc_spmm(y, rc_t, zeros_nph):
    k = functools.partial(
        pl.kernel,
        compiler_params=pltpu.CompilerParams(needs_layout_passes=False),
        out_type=jax.ShapeDtypeStruct((2, NP, H), f32),
        mesh=_mk_mesh(),
        scratch_types=[pltpu.VMEM((2, SUP, 2, K), i32),
                       pltpu.VMEM((ND, K, H), f32),
                       pltpu.VMEM_SHARED((NPZ, H), f32)]
        + [pltpu.SemaphoreType.DMA] * 10)(_sc_spmm_body)
    return k(y, rc_t, zeros_nph)


def _sc_edge(u, v, rc_t, zeros_nph):
    k = functools.partial(
        pl.kernel,
        compiler_params=pltpu.CompilerParams(needs_layout_passes=False),
        out_type=(jax.ShapeDtypeStruct((NT, NCH, K), f32),
                  jax.ShapeDtypeStruct((2, NP, H), f32)),
        mesh=_mk_mesh(),
        scratch_types=[pltpu.VMEM((NP,), f32), pltpu.VMEM((NP,), f32),
                       pltpu.VMEM((SUP, 2, K), i32), pltpu.VMEM((K,), f32),
                       pltpu.VMEM((K, H), f32),
                       pltpu.VMEM_SHARED((NPZ, H), f32)])(
        _sc_edge_body)
    return k(u, v, rc_t, zeros_nph)


def _sc_wspmm(y2, rcb, ew, zeros_nph):
    k = functools.partial(
        pl.kernel,
        compiler_params=pltpu.CompilerParams(needs_layout_passes=False),
        out_type=jax.ShapeDtypeStruct((2, NP, H), f32),
        mesh=_mk_mesh(),
        scratch_types=[pltpu.VMEM((2, SUPW, 2, K), i32),
                       pltpu.VMEM((2, SUPW, K), f32),
                       pltpu.VMEM((3, K, H), f32),
                       pltpu.VMEM_SHARED((NPZ, H), f32)]
        + [pltpu.SemaphoreType.DMA] * 12)(_sc_wspmm_body)
    return k(y2, rcb, ew, zeros_nph)


def kernel(x, W_feat, b_feat, W0, b0, W1, b1, W2, b2, We, be, Wn, bn_b,
           Wc, bc, Wo, bo, W1c, b1c, W2c, b2c, W1o, b1o, W2o, b2o,
           W1co, b1co, W2co, b2co, edge_index, batch):
    row = edge_index[0].astype(i32)
    col = edge_index[1].astype(i32)
    padn = jnp.full((NT * NCH * K - E,), N, i32)
    row_t = jnp.concatenate([row, padn]).reshape(NT, NCH, K)
    col_t = jnp.concatenate([col, padn]).reshape(NT, NCH, K)
    rc_t = jnp.stack([row_t, col_t], axis=2)
    xp = jnp.zeros((NP, x.shape[1]), f32).at[:N].set(x)
    zeros_nph = jnp.zeros((NP, H), f32)
    ones_kh = jnp.ones((K, H), f32)
    rowv = lambda a: a.reshape(1, H)
    bf = rowv(b_feat)
    We_p = jnp.zeros((2 * H, H), f32).at[:, :2].set(We)
    be_p = jnp.zeros((1, H), f32).at[0, :2].set(be)
    Wn_p = jnp.zeros((H, H), f32).at[:, :2].set(Wn)
    bnb_p = jnp.zeros((1, H), f32).at[0, :2].set(bn_b)
    bp = jnp.concatenate([batch.astype(i32), jnp.full((NP - N,), G, i32)])
    oh = (bp[:, None] == jnp.arange(G, dtype=i32)[None, :]).astype(f32)
    W2c_p = jnp.zeros((H, H), f32).at[:, :C].set(W2c)
    b2c_p = jnp.zeros((1, H), f32).at[0, :C].set(b2c)
    W2o_p = jnp.zeros((H, H), f32).at[:, :C].set(W2o)
    b2o_p = jnp.zeros((1, H), f32).at[0, :C].set(b2o)
    W2co_p = jnp.zeros((H, H), f32).at[:, :C].set(W2co)
    b2co_p = jnp.zeros((1, H), f32).at[0, :C].set(b2co)

    degu = _sc_hist(row_t, ones_kh, zeros_nph)
    sx = _tc_stats(xp)
    h, sh = _tc_feat(xp, sx, W_feat, bf)

    rc_sp = rc_t.reshape(NT * NCH, 2, K)
    for (Wi, bi) in ((W0, b0), (W1, b1), (W2, b2)):
        xw, y = _tc_front(h, sh, degu, Wi)
        z = _sc_spmm(y, rc_sp, zeros_nph)
        h, sh = _tc_back(z, xw, degu, rowv(bi))

    u, v, xc, xo, sxc, sxo = _tc_att(h, We_p, be_p, Wn_p, bnb_p)
    ec, degw = _sc_edge(u.reshape(NP), v.reshape(NP), rc_t, zeros_nph)

    xww, yw = _tc_wfront(xc, xo, sxc, sxo, degw, Wc, Wo)
    y2 = yw.reshape(2 * NP, H)
    NCHP = NSUPW * SUPW
    cpad = jnp.full((16, NCHP - NCHW, K), N, i32)
    row_w = jnp.concatenate([row_t.reshape(16, NCHW, K), cpad], axis=1)
    col_w = jnp.concatenate([col_t.reshape(16, NCHW, K), cpad], axis=1)
    rcb = jnp.stack([jnp.stack([row_w, col_w], axis=2),
                     jnp.stack([row_w + NP, col_w], axis=2)], axis=0)
    ew_w = jnp.concatenate(
        [ec.reshape(16, NCHW, K), jnp.zeros((16, NCHP - NCHW, K), f32)],
        axis=1)
    zw = _sc_wspmm(y2, rcb, ew_w, zeros_nph)

    xcg, xog = _tc_final(zw, xww, degw, rowv(bc), rowv(bo), oh)
    ws = (W1c, rowv(b1c), W2c_p, b2c_p, W1o, rowv(b1o), W2o_p, b2o_p,
          W1co[:H], W1co[H:], rowv(b1co), W2co_p, b2co_p)
    oc, oo, oco = _tc_readout(xcg, xog, ws)
    return (oc[:, :C], oo[:, :C], oco[:, :C])


# speedup vs baseline: 1.1649x; 1.0640x over previous
import functools

import jax
import jax.numpy as jnp
from jax import lax
from jax.experimental import pallas as pl
from jax.experimental.pallas import tpu as pltpu
from jax.experimental.pallas import tpu_sc as plsc

N = 10000
E = 320000
H = 128
C = 10
G = 128

NP = 10240
NPZ = 10112
K = 112
NCH = 90
NT = 32
NCHW = 2 * NCH
SUP = 6
SUPW = 8
NSUPW = 23
ND = 3
NCH0 = 132
NCH1 = 2 * NCH - NCH0
RB = 1024
NBLK = NP // RB
RPT = NP // 16
RPZ = NPZ // 16

f32 = jnp.float32
i32 = jnp.int32


def _rowmask(i):
    rows = lax.broadcasted_iota(i32, (RB, 1), 0) + i * RB
    return (rows < N).astype(f32)


def _affine(s_ref):
    s0 = s_ref[0:1, :]
    s1 = s_ref[1:2, :]
    m = s0 * (1.0 / N)
    v = s1 * (1.0 / N) - m * m
    al = lax.rsqrt(v + 1e-5)
    return al, -m * al + 1e-4


def _stats_of(x, i, s_ref):
    xm = x * _rowmask(i)
    part = jnp.concatenate(
        [jnp.sum(xm, 0, keepdims=True), jnp.sum(xm * xm, 0, keepdims=True),
         jnp.zeros((6, H), f32)], axis=0)

    @pl.when(i == 0)
    def _():
        s_ref[...] = part

    @pl.when(i > 0)
    def _():
        s_ref[...] = s_ref[...] + part


def _tc_stats_body(x_ref, s_ref):
    _stats_of(x_ref[...], pl.program_id(0), s_ref)


def _tc_feat_body(x_ref, sx_ref, Wf_ref, bf_ref, h_ref, s1_ref):
    i = pl.program_id(0)
    al, bt = _affine(sx_ref)
    xn = x_ref[...] * al + bt
    h = jnp.maximum(jnp.dot(xn, Wf_ref[...], preferred_element_type=f32)
                    + bf_ref[...], 0.0)
    h_ref[...] = h
    _stats_of(h, i, s1_ref)


def _lane_bcast(x32, lane):
    ri = lax.broadcasted_iota(i32, (32, H), 0)
    return jnp.dot(x32, (ri == lane).astype(f32), preferred_element_type=f32)


def _dis_u(deg_ref):
    return lax.rsqrt(_lane_bcast(deg_ref[0] + deg_ref[1], 0) + 1.0)


def _tc_front_body(h_ref, sh_ref, deg_ref, W_ref, xw_ref, y_ref):
    i = pl.program_id(0)
    al, bt = _affine(sh_ref)
    xn = h_ref[...] * al + bt
    xw = jnp.dot(xn, W_ref[...], preferred_element_type=f32)
    dis = _dis_u(deg_ref)
    xw_ref[...] = xw
    y_ref[...] = dis * xw * _rowmask(i)


def _tc_back_body(z_ref, xw_ref, deg_ref, b_ref, h_ref, s_ref):
    i = pl.program_id(0)
    dis = _dis_u(deg_ref)
    xw = xw_ref[...]
    h = jnp.maximum(dis * (z_ref[0] + z_ref[1]) + dis * dis * xw + b_ref[...],
                    0.0)
    h_ref[...] = h
    _stats_of(h, i, s_ref)


def _tc_att_body(h_ref, We_ref, be_ref, Wn_ref, bnb_ref,
                 u_ref, v_ref, xc_ref, xo_ref, sc_ref, so_ref):
    i = pl.program_id(0)
    hb = h_ref[...]
    lanes = lax.broadcasted_iota(i32, (1, H), 1)
    sel = jnp.where(lanes == 0, 1.0, jnp.where(lanes == 1, -1.0, 0.0))
    wuv = lax.dot_general(sel, We_ref[...], (((1,), (1,)), ((), ())),
                          preferred_element_type=f32)
    wu = wuv[:, 0:H]
    wv = wuv[:, H:2 * H]
    be_d = jnp.sum(be_ref[...] * sel, axis=1, keepdims=True)
    urows = []
    vrows = []
    for sb in range(RB // 128):
        hs = hb[sb * 128:(sb + 1) * 128, :]
        urows.append(lax.dot_general(wu, hs, (((1,), (1,)), ((), ())),
                                     preferred_element_type=f32) + be_d)
        vrows.append(lax.dot_general(wv, hs, (((1,), (1,)), ((), ())),
                                     preferred_element_type=f32))
    u_ref[...] = jnp.concatenate(urows, axis=0)
    v_ref[...] = jnp.concatenate(vrows, axis=0)
    nl = jnp.dot(hb, Wn_ref[...], preferred_element_type=f32) + bnb_ref[...]
    d01 = nl[:, 0:1] - nl[:, 1:2]
    att0 = 1.0 / (1.0 + jnp.exp(-d01))
    xc = att0 * hb
    xo = hb - xc
    xc_ref[...] = xc
    xo_ref[...] = xo
    _stats_of(xc, i, sc_ref)
    _stats_of(xo, i, so_ref)


def _split_dis(dgw_ref):
    dsum = dgw_ref[0] + dgw_ref[1]
    return (lax.rsqrt(_lane_bcast(dsum, 0) + 1.0),
            lax.rsqrt(_lane_bcast(dsum, 16) + 1.0))


def _tc_wfront_body(xc_ref, xo_ref, sc_ref, so_ref, dgw_ref,
                    Wc_ref, Wo_ref, xw_ref, y_ref):
    i = pl.program_id(0)
    msk = _rowmask(i)
    disc, diso = _split_dis(dgw_ref)
    alc, btc = _affine(sc_ref)
    xwc = jnp.dot(xc_ref[...] * alc + btc, Wc_ref[...],
                  preferred_element_type=f32)
    alo, bto = _affine(so_ref)
    xwo = jnp.dot(xo_ref[...] * alo + bto, Wo_ref[...],
                  preferred_element_type=f32)
    xw_ref[0] = xwc
    xw_ref[1] = xwo
    y_ref[0] = disc * xwc * msk
    y_ref[1] = diso * xwo * msk


def _tc_final_body(zw_ref, xw_ref, dgw_ref, bc_ref, bo_ref, oh_ref,
                   pc_ref, po_ref):
    i = pl.program_id(0)
    disc, diso = _split_dis(dgw_ref)
    xc2 = jnp.maximum(disc * zw_ref[0] + disc * disc * xw_ref[0] + bc_ref[...],
                      0.0)
    xo2 = jnp.maximum(diso * zw_ref[1] + diso * diso * xw_ref[1] + bo_ref[...],
                      0.0)
    oh = oh_ref[...]
    pc = lax.dot_general(oh, xc2, (((0,), (0,)), ((), ())),
                         preferred_element_type=f32)
    po = lax.dot_general(oh, xo2, (((0,), (0,)), ((), ())),
                         preferred_element_type=f32)

    @pl.when(i == 0)
    def _():
        pc_ref[...] = pc
        po_ref[...] = po

    @pl.when(i > 0)
    def _():
        pc_ref[...] = pc_ref[...] + pc
        po_ref[...] = po_ref[...] + po


def _gstats(x):
    s0 = jnp.sum(x, 0, keepdims=True)
    s1 = jnp.sum(x * x, 0, keepdims=True)
    m = s0 * (1.0 / G)
    v = s1 * (1.0 / G) - m * m
    al = lax.rsqrt(v + 1e-5)
    return al, -m * al + 1e-4


def _logsm(lo):
    lanes = lax.broadcasted_iota(i32, (G, H), 1)
    lom = jnp.where(lanes < C, lo, -1e30)
    mx = jnp.max(lom, axis=1, keepdims=True)
    ls = jnp.log(jnp.sum(jnp.exp(lom - mx), axis=1, keepdims=True))
    return lo - mx - ls


def _tc_readout_body(xcg_ref, xog_ref, W1c_ref, b1c_ref, W2c_ref, b2c_ref,
                     W1o_ref, b1o_ref, W2o_ref, b2o_ref,
                     W1t_ref, W1b_ref, b1co_ref, W2co_ref, b2co_ref,
                     oc_ref, oo_ref, oco_ref):
    xcg = xcg_ref[...]
    xog = xog_ref[...]

    def head(xg, W1r, b1r, W2r, b2r):
        al, bt = _gstats(xg)
        hh = jnp.maximum(jnp.dot(xg * al + bt, W1r,
                                 preferred_element_type=f32) + b1r, 0.0)
        al2, bt2 = _gstats(hh)
        return _logsm(jnp.dot(hh * al2 + bt2, W2r,
                              preferred_element_type=f32) + b2r)

    oc_ref[...] = head(xcg, W1c_ref[...], b1c_ref[...], W2c_ref[...],
                       b2c_ref[...])
    oo_ref[...] = head(xog, W1o_ref[...], b1o_ref[...], W2o_ref[...],
                       b2o_ref[...])
    alc, btc = _gstats(xcg)
    alo, bto = _gstats(xog)
    hh = jnp.maximum(
        jnp.dot(xcg * alc + btc, W1t_ref[...], preferred_element_type=f32)
        + jnp.dot(xog * alo + bto, W1b_ref[...], preferred_element_type=f32)
        + b1co_ref[...], 0.0)
    al2, bt2 = _gstats(hh)
    oco_ref[...] = _logsm(jnp.dot(hh * al2 + bt2, W2co_ref[...],
                                  preferred_element_type=f32) + b2co_ref[...])


_B_NH = pl.BlockSpec((RB, H), lambda i: (i, 0))
_B_2NH = pl.BlockSpec((2, RB, H), lambda i: (0, i, 0))
_B_DG = pl.BlockSpec((2, RB, 32), lambda i: (0, i, 0))
_B_S = pl.BlockSpec((8, H), lambda i: (0, 0))
_B_W = pl.BlockSpec((H, H), lambda i: (0, 0))
_B_B = pl.BlockSpec((1, H), lambda i: (0, 0))
_B_U = pl.BlockSpec((RB // 128, H), lambda i: (i, 0))
_B_G = pl.BlockSpec((G, H), lambda i: (0, 0))
_SNH = jax.ShapeDtypeStruct((NP, H), f32)
_S2NH = jax.ShapeDtypeStruct((2, NP, H), f32)
_SS = jax.ShapeDtypeStruct((8, H), f32)
_SU = jax.ShapeDtypeStruct((NP // H, H), f32)
_SG = jax.ShapeDtypeStruct((G, H), f32)


def _tc_stats(x):
    return pl.pallas_call(_tc_stats_body, grid=(NBLK,), in_specs=[_B_NH],
                          out_specs=_B_S, out_shape=_SS)(x)


def _tc_feat(x, sx, Wf, bf):
    return pl.pallas_call(
        _tc_feat_body, grid=(NBLK,),
        in_specs=[_B_NH, _B_S, _B_W, _B_B],
        out_specs=[_B_NH, _B_S], out_shape=[_SNH, _SS])(x, sx, Wf, bf)


def _tc_front(h, sh, degb, W):
    return pl.pallas_call(
        _tc_front_body, grid=(NBLK,),
        in_specs=[_B_NH, _B_S, _B_DG, _B_W],
        out_specs=[_B_NH, _B_NH], out_shape=[_SNH, _SNH])(h, sh, degb, W)


def _tc_back(z, xw, degb, b):
    return pl.pallas_call(
        _tc_back_body, grid=(NBLK,),
        in_specs=[_B_2NH, _B_NH, _B_DG, _B_B],
        out_specs=[_B_NH, _B_S], out_shape=[_SNH, _SS])(z, xw, degb, b)


def _tc_att(h, We_p, be_p, Wn_p, bnb_p):
    return pl.pallas_call(
        _tc_att_body, grid=(NBLK,),
        in_specs=[_B_NH, pl.BlockSpec((2 * H, H), lambda i: (0, 0)), _B_B,
                  _B_W, _B_B],
        out_specs=[_B_U, _B_U, _B_NH, _B_NH, _B_S, _B_S],
        out_shape=[_SU, _SU, _SNH, _SNH, _SS, _SS])(h, We_p, be_p, Wn_p, bnb_p)


def _tc_wfront(xc, xo, sc, so, dgw, Wc, Wo):
    return pl.pallas_call(
        _tc_wfront_body, grid=(NBLK,),
        in_specs=[_B_NH, _B_NH, _B_S, _B_S, _B_DG, _B_W, _B_W],
        out_specs=[_B_2NH, _B_2NH],
        out_shape=[_S2NH, _S2NH])(xc, xo, sc, so, dgw, Wc, Wo)


def _tc_final(zw, xw, dgw, bc, bo, oh):
    return pl.pallas_call(
        _tc_final_body, grid=(NBLK,),
        in_specs=[_B_2NH, _B_2NH, _B_DG, _B_B, _B_B,
                  pl.BlockSpec((RB, G), lambda i: (i, 0))],
        out_specs=[_B_G, _B_G], out_shape=[_SG, _SG])(zw, xw, dgw, bc, bo, oh)


def _tc_readout(xcg, xog, ws):
    gspec = pl.BlockSpec((G, H), lambda: (0, 0))
    bspec = pl.BlockSpec((1, H), lambda: (0, 0))
    specs = [gspec, gspec] + [gspec if w.shape[0] == H else bspec for w in ws]
    return pl.pallas_call(
        _tc_readout_body, grid=(),
        in_specs=specs, out_specs=[gspec, gspec, gspec],
        out_shape=[_SG, _SG, _SG])(xcg, xog, *ws)


def _sc_hist_body(row_hbm, ones_hbm, zeros_hbm, deg_out,
                  row_v, ones_v, deg_sh):
    cid = lax.axis_index("c")
    sid = lax.axis_index("s")
    wid = cid * 16 + sid
    r0 = sid * RPT
    pltpu.sync_copy(row_hbm.at[wid], row_v)
    pltpu.sync_copy(ones_hbm, ones_v)
    pltpu.sync_copy(zeros_hbm.at[pl.ds(r0, RPT)], deg_sh.at[pl.ds(r0, RPT)])
    plsc.subcore_barrier()

    def step(jb, c):
        pltpu.sync_copy(ones_v, deg_sh.at[row_v.at[jb]], add=True)
        return c

    lax.fori_loop(0, NCH, step, 0)
    plsc.subcore_barrier()
    pltpu.sync_copy(deg_sh.at[pl.ds(r0, RPT)], deg_out.at[cid, pl.ds(r0, RPT)])


def _sc_hist(row_t, ones_k32, zeros_np32):
    k = functools.partial(
        pl.kernel,
        compiler_params=pltpu.CompilerParams(needs_layout_passes=False),
        out_type=jax.ShapeDtypeStruct((2, NP, 32), f32),
        mesh=_mk_mesh(),
        scratch_types=[pltpu.VMEM((NCH, K), i32), pltpu.VMEM((K, 32), f32),
                       pltpu.VMEM_SHARED((NP, 32), f32)])(_sc_hist_body)
    return k(row_t, ones_k32, zeros_np32)


def _sc_spmm_body(y_hbm, rc_hbm, zeros_hbm, z_out, IS, D, z_sh,
                  i0, i1, g0, g1, g2, g3, s0, s1, s2, s3):
    cid = lax.axis_index("c")
    sid = lax.axis_index("s")
    r0 = sid * RPZ
    isems = (i0, i1)
    gsems = (g0, g1, g2, g3)
    ssems = (s0, s1, s2, s3)

    def pipeline(nch, start):
        nsup = nch // SUP

        def idx_dma(s):
            pltpu.async_copy(rc_hbm.at[pl.ds(start + s * SUP, SUP)],
                             IS.at[s % 2], isems[s % 2])

        def wait_i(s):
            pltpu.make_async_copy(rc_hbm.at[pl.ds(0, SUP)], IS.at[s % 2],
                                  isems[s % 2]).wait()

        def gather(c, slot):
            s, j = divmod(c, SUP)
            pltpu.async_copy(y_hbm.at[IS.at[s % 2, j, 0]], D.at[slot],
                             gsems[slot])

        def wait_g(slot):
            pltpu.make_async_copy(y_hbm.at[IS.at[0, 0, 0]], D.at[slot],
                                  gsems[slot]).wait()

        def scatter(c):
            s, j = divmod(c, SUP)
            pltpu.async_copy(D.at[c % ND], z_sh.at[IS.at[s % 2, j, 1]],
                             ssems[c % ND], add=True)

        def wait_s(slot):
            pltpu.make_async_copy(D.at[slot], z_sh.at[IS.at[0, 0, 1]],
                                  ssems[slot]).wait()

        idx_dma(0)
        wait_i(0)
        gather(0, 0)
        gather(1, 1)
        swaited = set()

        def scatter_done(cc):
            if cc >= 0 and cc not in swaited:
                wait_s(cc % ND)
                swaited.add(cc)

        for c in range(nch):
            s, j = divmod(c, SUP)
            if j == 0 and s + 1 < nsup:
                scatter_done(c - 2)
                scatter_done(c - 1)
                idx_dma(s + 1)
            if j == SUP - 2 and s + 1 < nsup:
                wait_i(s + 1)
            scatter_done(c + 2 - ND)
            gather(min(c + 2, nch - 1), (c + 2) % ND)
            wait_g(c % ND)
            scatter(c)
        wait_g(nch % ND)
        wait_g((nch + 1) % ND)
        scatter_done(nch - 2)
        scatter_done(nch - 1)

    pltpu.sync_copy(zeros_hbm.at[pl.ds(r0, RPZ)], z_sh.at[pl.ds(r0, RPZ)])
    plsc.subcore_barrier()

    @pl.when(cid == 0)
    def _():
        pipeline(NCH0, sid * NCH0)

    @pl.when(cid == 1)
    def _():
        pipeline(NCH1, 16 * NCH0 + sid * NCH1)

    plsc.subcore_barrier()
    pltpu.sync_copy(z_sh.at[pl.ds(r0, RPZ)], z_out.at[cid, pl.ds(r0, RPZ)])

    @pl.when(sid == 15)
    def _():
        pltpu.sync_copy(zeros_hbm.at[pl.ds(0, NP - NPZ)],
                        z_out.at[cid, pl.ds(NPZ, NP - NPZ)])


def _sc_edge_body(u_hbm, v_hbm, rc_hbm, zeros_hbm, ec_out, degw_out,
                  u_v, v_v, IS, ecb, RW, deg_sh):
    cid = lax.axis_index("c")
    sid = lax.axis_index("s")
    wid = cid * 16 + sid
    r0 = sid * RPZ
    pltpu.sync_copy(u_hbm, u_v)
    pltpu.sync_copy(v_hbm, v_v)
    pltpu.sync_copy(zeros_hbm.at[pl.ds(r0, RPZ)], deg_sh.at[pl.ds(r0, RPZ)])
    plsc.subcore_barrier()

    def chunk(c, carry):
        s = c // SUP
        j = c - s * SUP

        @pl.when(j == 0)
        def _():
            pltpu.sync_copy(rc_hbm.at[wid, pl.ds(s * SUP, SUP)], IS)

        for g in range(K // 16):
            r16 = IS[j, 0, pl.ds(g * 16, 16)]
            c16 = IS[j, 1, pl.ds(g * 16, 16)]
            uu = plsc.load_gather(u_v, [r16])
            vv = plsc.load_gather(v_v, [c16])
            ec = 1.0 / (1.0 + jnp.exp(-(uu + vv)))
            ecb[pl.ds(g * 16, 16)] = ec

        def edge(e, cc):
            w16 = plsc.load_gather(ecb, [jnp.full((16,), e, i32)])
            RW[e, pl.ds(0, 16)] = w16
            RW[e, pl.ds(16, 16)] = 1.0 - w16
            return cc

        lax.fori_loop(0, K, edge, 0)
        pltpu.sync_copy(ecb, ec_out.at[wid, c])
        pltpu.sync_copy(RW, deg_sh.at[IS.at[j, 0]], add=True)
        return carry

    lax.fori_loop(0, NCH, chunk, 0)
    plsc.subcore_barrier()
    pltpu.sync_copy(deg_sh.at[pl.ds(r0, RPZ)], degw_out.at[cid, pl.ds(r0, RPZ)])

    @pl.when(sid == 15)
    def _():
        pltpu.sync_copy(zeros_hbm.at[pl.ds(0, NP - NPZ)],
                        degw_out.at[cid, pl.ds(NPZ, NP - NPZ)])


def _sc_wspmm_body(y_hbm, rcb_hbm, ew_hbm, zeros_hbm, z_out, IS, EW, D, z_sh,
                   i0, i1, e0, e1, g0, g1, g2, g3, s0, s1, s2, s3):
    cid = lax.axis_index("c")
    sid = lax.axis_index("s")
    r0 = sid * RPZ
    isems = (i0, i1)
    esems = (e0, e1)
    gsems = (g0, g1, g2, g3)
    ssems = (s0, s1, s2, s3)
    fv = jnp.full((16,), cid.astype(f32), f32)
    a0 = fv
    a1 = 1.0 - 2.0 * fv

    def idx_dma(sb, s):
        pltpu.async_copy(rcb_hbm.at[cid, sid, pl.ds(s * SUPW, SUPW)],
                         IS.at[sb], isems[sb])
        pltpu.async_copy(ew_hbm.at[sid, pl.ds(s * SUPW, SUPW)],
                         EW.at[sb], esems[sb])

    def wait_i(sb):
        pltpu.make_async_copy(rcb_hbm.at[0, 0, pl.ds(0, SUPW)], IS.at[sb],
                              isems[sb]).wait()
        pltpu.make_async_copy(ew_hbm.at[0, pl.ds(0, SUPW)], EW.at[sb],
                              esems[sb]).wait()

    def gather(slot, s2, j):
        pltpu.async_copy(y_hbm.at[IS.at[s2, j, 0]], D.at[slot], gsems[slot])

    def wait_g(slot):
        pltpu.make_async_copy(y_hbm.at[IS.at[0, 0, 0]], D.at[slot],
                              gsems[slot]).wait()

    def scatter(b, s2, j):
        pltpu.async_copy(D.at[b], z_sh.at[IS.at[s2, j, 1]], ssems[b],
                         add=True)

    def wait_s(slot):
        pltpu.make_async_copy(D.at[slot], z_sh.at[IS.at[0, 0, 1]],
                              ssems[slot]).wait()

    idx_dma(0, 0)
    pltpu.sync_copy(zeros_hbm.at[pl.ds(r0, RPZ)], z_sh.at[pl.ds(r0, RPZ)])
    plsc.subcore_barrier()
    wait_i(0)
    gather(0, 0, 0)
    gather(1, 0, 1)

    def triple(t, carry):
        c0 = 3 * t
        for b in range(3):
            c = c0 + b
            s = c // SUPW
            j = c - s * SUPW
            s2 = s % 2
            if b >= 1:
                wait_s((b + 2) % 3)
            else:
                @pl.when(t > 0)
                def _():
                    wait_s(2)
            for kk in range(2):
                @pl.when((j == 0) & (s + 1 < NSUPW) & ((s + 1) % 2 == kk))
                def _():
                    idx_dma(kk, s + 1)
            for kk in range(2):
                @pl.when((j == SUPW - 2) & (s + 1 < NSUPW)
                         & ((s + 1) % 2 == kk))
                def _():
                    wait_i(kk)
            p = jnp.minimum(c + 2, NCHW - 1)
            sp = p // SUPW
            gather((b + 2) % 3, sp % 2, p - sp * SUPW)
            wait_g(b)
            sv = jnp.full((16,), s2, i32)
            jv = jnp.full((16,), j, i32)

            def rbody(r, cc):
                w16 = plsc.load_gather(EW, [sv, jv, jnp.full((16,), r, i32)])
                w16 = a0 + a1 * w16
                for fch in range(8):
                    sl = D[b, r, pl.ds(fch * 16, 16)]
                    D[b, r, pl.ds(fch * 16, 16)] = sl * w16
                return cc

            lax.fori_loop(0, K, rbody, 0)
            scatter(b, s2, j)
        return carry

    lax.fori_loop(0, NCHW // 3, triple, 0)
    wait_g(0)
    wait_g(1)
    wait_s(2)
    plsc.subcore_barrier()
    pltpu.sync_copy(z_sh.at[pl.ds(r0, RPZ)], z_out.at[cid, pl.ds(r0, RPZ)])

    @pl.when(sid == 15)
    def _():
        pltpu.sync_copy(zeros_hbm.at[pl.ds(0, NP - NPZ)],
                        z_out.at[cid, pl.ds(NPZ, NP - NPZ)])


def _mk_mesh():
    return plsc.VectorSubcoreMesh(core_axis_name="c", subcore_axis_name="s")


def _sc_spmm(y, rc_t, zeros_nph):
    k = functools.partial(
        pl.kernel,
        compiler_params=pltpu.CompilerParams(needs_layout_passes=False),
        out_type=jax.ShapeDtypeStruct((2, NP, H), f32),
        mesh=_mk_mesh(),
        scratch_types=[pltpu.VMEM((2, SUP, 2, K), i32),
                       pltpu.VMEM((ND, K, H), f32),
                       pltpu.VMEM_SHARED((NPZ, H), f32)]
        + [pltpu.SemaphoreType.DMA] * 10)(_sc_spmm_body)
    return k(y, rc_t, zeros_nph)


def _sc_edge(u, v, rc_t, zeros_np32):
    k = functools.partial(
        pl.kernel,
        compiler_params=pltpu.CompilerParams(needs_layout_passes=False),
        out_type=(jax.ShapeDtypeStruct((NT, NCH, K), f32),
                  jax.ShapeDtypeStruct((2, NP, 32), f32)),
        mesh=_mk_mesh(),
        scratch_types=[pltpu.VMEM((NP,), f32), pltpu.VMEM((NP,), f32),
                       pltpu.VMEM((SUP, 2, K), i32), pltpu.VMEM((K,), f32),
                       pltpu.VMEM((K, 32), f32),
                       pltpu.VMEM_SHARED((NPZ, 32), f32)])(
        _sc_edge_body)
    return k(u, v, rc_t, zeros_np32)


def _sc_wspmm(y2, rcb, ew, zeros_nph):
    k = functools.partial(
        pl.kernel,
        compiler_params=pltpu.CompilerParams(needs_layout_passes=False),
        out_type=jax.ShapeDtypeStruct((2, NP, H), f32),
        mesh=_mk_mesh(),
        scratch_types=[pltpu.VMEM((2, SUPW, 2, K), i32),
                       pltpu.VMEM((2, SUPW, K), f32),
                       pltpu.VMEM((3, K, H), f32),
                       pltpu.VMEM_SHARED((NPZ, H), f32)]
        + [pltpu.SemaphoreType.DMA] * 12)(_sc_wspmm_body)
    return k(y2, rcb, ew, zeros_nph)


def kernel(x, W_feat, b_feat, W0, b0, W1, b1, W2, b2, We, be, Wn, bn_b,
           Wc, bc, Wo, bo, W1c, b1c, W2c, b2c, W1o, b1o, W2o, b2o,
           W1co, b1co, W2co, b2co, edge_index, batch):
    row = edge_index[0].astype(i32)
    col = edge_index[1].astype(i32)
    padn = jnp.full((NT * NCH * K - E,), N, i32)
    row_t = jnp.concatenate([row, padn]).reshape(NT, NCH, K)
    col_t = jnp.concatenate([col, padn]).reshape(NT, NCH, K)
    rc_t = jnp.stack([row_t, col_t], axis=2)
    xp = jnp.zeros((NP, x.shape[1]), f32).at[:N].set(x)
    zeros_nph = jnp.zeros((NP, H), f32)
    zeros_np32 = jnp.zeros((NP, 32), f32)
    ones_k32 = jnp.ones((K, 32), f32)
    rowv = lambda a: a.reshape(1, H)
    bf = rowv(b_feat)
    We_p = jnp.zeros((2 * H, H), f32).at[:, :2].set(We)
    be_p = jnp.zeros((1, H), f32).at[0, :2].set(be)
    Wn_p = jnp.zeros((H, H), f32).at[:, :2].set(Wn)
    bnb_p = jnp.zeros((1, H), f32).at[0, :2].set(bn_b)
    bp = jnp.concatenate([batch.astype(i32), jnp.full((NP - N,), G, i32)])
    oh = (bp[:, None] == jnp.arange(G, dtype=i32)[None, :]).astype(f32)
    W2c_p = jnp.zeros((H, H), f32).at[:, :C].set(W2c)
    b2c_p = jnp.zeros((1, H), f32).at[0, :C].set(b2c)
    W2o_p = jnp.zeros((H, H), f32).at[:, :C].set(W2o)
    b2o_p = jnp.zeros((1, H), f32).at[0, :C].set(b2o)
    W2co_p = jnp.zeros((H, H), f32).at[:, :C].set(W2co)
    b2co_p = jnp.zeros((1, H), f32).at[0, :C].set(b2co)

    degu = _sc_hist(row_t, ones_k32, zeros_np32)
    sx = _tc_stats(xp)
    h, sh = _tc_feat(xp, sx, W_feat, bf)

    rc_sp = rc_t.reshape(NT * NCH, 2, K)
    for (Wi, bi) in ((W0, b0), (W1, b1), (W2, b2)):
        xw, y = _tc_front(h, sh, degu, Wi)
        z = _sc_spmm(y, rc_sp, zeros_nph)
        h, sh = _tc_back(z, xw, degu, rowv(bi))

    u, v, xc, xo, sxc, sxo = _tc_att(h, We_p, be_p, Wn_p, bnb_p)
    ec, degw = _sc_edge(u.reshape(NP), v.reshape(NP), rc_t, zeros_np32)

    xww, yw = _tc_wfront(xc, xo, sxc, sxo, degw, Wc, Wo)
    y2 = yw.reshape(2 * NP, H)
    NCHP = NSUPW * SUPW
    cpad = jnp.full((16, NCHP - NCHW, K), N, i32)
    row_w = jnp.concatenate([row_t.reshape(16, NCHW, K), cpad], axis=1)
    col_w = jnp.concatenate([col_t.reshape(16, NCHW, K), cpad], axis=1)
    rcb = jnp.stack([jnp.stack([row_w, col_w], axis=2),
                     jnp.stack([row_w + NP, col_w], axis=2)], axis=0)
    ew_w = jnp.concatenate(
        [ec.reshape(16, NCHW, K), jnp.zeros((16, NCHP - NCHW, K), f32)],
        axis=1)
    zw = _sc_wspmm(y2, rcb, ew_w, zeros_nph)

    xcg, xog = _tc_final(zw, xww, degw, rowv(bc), rowv(bo), oh)
    ws = (W1c, rowv(b1c), W2c_p, b2c_p, W1o, rowv(b1o), W2o_p, b2o_p,
          W1co[:H], W1co[H:], rowv(b1co), W2co_p, b2co_p)
    oc, oo, oco = _tc_readout(xcg, xog, ws)
    return (oc[:, :C], oo[:, :C], oco[:, :C])
